# Initial kernel scaffold; baseline (speedup 1.0000x reference)
import jax
import jax.numpy as jnp
from jax.experimental import pallas as pl


def kernel(num_prop_burst, cat_prop_burst, tweet_range_list, edge_index_burst, re_index, des, tweet, num_prop, cat_prop, edge_index_rgcn, edge_type, params):
    raise NotImplementedError("write your pallas kernel here")


# speedup vs baseline: 1.0011x; 1.0011x over previous
import functools

import jax
import jax.numpy as jnp
from jax import lax
from jax.experimental import pallas as pl
from jax.experimental.pallas import tpu as pltpu
from jax.experimental.pallas import tpu_sc as plsc

_EPS_FA = 0.1

_NC = 2
_NS = 16
_NW = _NC * _NS


_SC_PARAMS = pltpu.CompilerParams(use_tc_tiling_on_sc=False)


def _vmesh():
    return plsc.VectorSubcoreMesh(
        core_axis_name="c", subcore_axis_name="s",
        num_cores=_NC, num_subcores=_NS)


def _wid():
    return lax.axis_index("s") * _NC + lax.axis_index("c")


def _sc_gather_rows(table, idx, *, chunk=80):
    n, d = table.shape
    b = idx.shape[0]
    per_w = -(-b // (_NW * chunk)) * chunk
    bp = per_w * _NW
    nchunk = per_w // chunk
    idx_p = jnp.concatenate([idx, jnp.zeros((bp - b,), jnp.int32)])
    idx2 = idx_p.reshape(_NW * nchunk, chunk)

    @functools.partial(
        pl.kernel,
        out_type=jax.ShapeDtypeStruct((bp, d), jnp.float32),
        mesh=_vmesh(),
        scratch_types=[
            pltpu.VMEM((nchunk, chunk), jnp.int32),
            pltpu.VMEM((chunk, d), jnp.float32),
            pltpu.SemaphoreType.DMA,
        ],
        compiler_params=_SC_PARAMS,
    )
    def k(table_hbm, idx_hbm, out_hbm, idx_v, rows_v, sem):
        w = _wid()
        pltpu.sync_copy(idx_hbm.at[pl.ds(w * nchunk, nchunk)], idx_v)

        @pl.loop(0, nchunk)
        def _(j):
            pltpu.async_copy(table_hbm.at[idx_v.at[j]], rows_v, sem).wait()
            pltpu.sync_copy(
                rows_v, out_hbm.at[pl.ds(w * per_w + j * chunk, chunk)])

    return k(table, idx2)[:b]


def _faconv(x, x0, edge_index, w_l, w_r):
    src, dst = edge_index[0], edge_index[1]
    n = x.shape[0]
    deg = jnp.zeros((n,), x.dtype).at[dst].add(jnp.ones(dst.shape, x.dtype))
    dis = jnp.where(deg > 0, lax.rsqrt(jnp.maximum(deg, 1.0)), 0.0)
    ew = dis[src] * dis[dst]
    alpha = jnp.tanh((x @ w_l)[src] + (x @ w_r)[dst])
    out = jnp.zeros_like(x).at[dst].add(x[src] * (alpha * ew)[:, None])
    return out + _EPS_FA * x0


def _rgcn(x, edge_index, edge_type, w, root, b):
    src, dst = edge_index[0], edge_index[1]
    n, d = x.shape
    out = x @ root + b
    for r in range(w.shape[0]):
        m = (edge_type == r).astype(x.dtype)
        s = jnp.zeros((n, d), x.dtype).at[dst].add(x[src] * m[:, None])
        c = jnp.zeros((n,), x.dtype).at[dst].add(m)
        out = out + (s / jnp.maximum(c, 1.0)[:, None]) @ w[r]
    return out


def kernel(num_prop_burst, cat_prop_burst, tweet_range_list, edge_index_burst,
           re_index, des, tweet, num_prop, cat_prop, edge_index_rgcn,
           edge_type, params):
    p = params
    act = jax.nn.leaky_relu

    num = act(num_prop_burst @ p['w_num'] + p['b_num'])
    cat = act(cat_prop_burst @ p['w_cat'] + p['b_cat'])
    x = jnp.concatenate([num, cat], axis=1)
    x = act(x @ p['w_tog'] + p['b_tog'])

    x1 = _faconv(x, x, edge_index_burst, p['w_att_l'], p['w_att_r'])
    x2 = _faconv(x1, x, edge_index_burst, p['w_att_l'], p['w_att_r'])
    x2 = (x2 ** 2 + 1e-08) ** 0.5

    num0 = x2.shape[1]
    nseg = tweet_range_list.shape[0] - 1
    pos = jnp.arange(x2.shape[0], dtype=tweet_range_list.dtype)
    seg = jnp.searchsorted(tweet_range_list, pos, side='right') - 1
    seg = jnp.where((seg >= 0) & (seg < nseg), seg, nseg)
    x3 = jax.ops.segment_sum(x2, seg, num_segments=nseg + 1)[:nseg]
    num_users = des.shape[0]
    x3 = jnp.concatenate(
        [x3, jnp.zeros((num_users - nseg, num0), x3.dtype)], axis=0)

    x3 = _sc_gather_rows(x3, re_index)
    x_burst = act(x3 @ p['w_map'] + p['b_map'])

    d = act(des @ p['w_des'] + p['b_des'])
    t = act(tweet @ p['w_tw'] + p['b_tw'])
    n = act(num_prop @ p['w_np'] + p['b_np'])
    c = act(cat_prop @ p['w_cp'] + p['b_cp'])
    xr = jnp.concatenate([d, t, n, c], axis=1)
    xr = act(xr @ p['w_in'] + p['b_in'])
    xr = _rgcn(xr, edge_index_rgcn, edge_type, p['w_rgcn'], p['w_root'],
               p['b_rgcn'])
    xr = _rgcn(xr, edge_index_rgcn, edge_type, p['w_rgcn'], p['w_root'],
               p['b_rgcn'])
    x_rgcn = act(xr @ p['w_out1'] + p['b_out1'])

    xcat = jnp.concatenate([x_burst, x_rgcn], axis=1)
    xcat = act(xcat @ p['w_f0'] + p['b_f0'])
    return xcat @ p['w_f'] + p['b_f']


# speedup vs baseline: 1.0119x; 1.0108x over previous
import functools

import jax
import jax.numpy as jnp
from jax import lax
from jax.experimental import pallas as pl
from jax.experimental.pallas import tpu as pltpu
from jax.experimental.pallas import tpu_sc as plsc

_EPS_FA = 0.1

_NC = 2
_NS = 16
_NW = _NC * _NS


_SC_PARAMS = pltpu.CompilerParams(use_tc_tiling_on_sc=False)


def _vmesh():
    return plsc.VectorSubcoreMesh(
        core_axis_name="c", subcore_axis_name="s",
        num_cores=_NC, num_subcores=_NS)


def _wid():
    return lax.axis_index("s") * _NC + lax.axis_index("c")


def _sc_gather_rows(table, idx, *, chunk=80):
    n, d = table.shape
    b = idx.shape[0]
    per_w = -(-b // (_NW * chunk)) * chunk
    bp = per_w * _NW
    nchunk = per_w // chunk
    idx_p = jnp.concatenate([idx, jnp.zeros((bp - b,), jnp.int32)])
    idx2 = idx_p.reshape(_NW * nchunk, chunk)

    @functools.partial(
        pl.kernel,
        out_type=jax.ShapeDtypeStruct((bp, d), jnp.float32),
        mesh=_vmesh(),
        scratch_types=[
            pltpu.VMEM((nchunk, chunk), jnp.int32),
            pltpu.VMEM((chunk, d), jnp.float32),
            pltpu.SemaphoreType.DMA,
        ],
        compiler_params=_SC_PARAMS,
    )
    def k(table_hbm, idx_hbm, out_hbm, idx_v, rows_v, sem):
        w = _wid()
        pltpu.sync_copy(idx_hbm.at[pl.ds(w * nchunk, nchunk)], idx_v)

        @pl.loop(0, nchunk)
        def _(j):
            pltpu.async_copy(table_hbm.at[idx_v.at[j]], rows_v, sem).wait()
            pltpu.sync_copy(
                rows_v, out_hbm.at[pl.ds(w * per_w + j * chunk, chunk)])

    return k(table, idx2)[:b]


def _sc_count(idx2d, nr):
    etot = idx2d.shape[0] * 128
    per_sc = etot // 2
    per_tile = per_sc // _NS
    ngrp = per_tile // 1024
    rpt = nr // _NS
    zeros = jnp.zeros((nr, 16), jnp.float32)

    @functools.partial(
        pl.kernel,
        out_type=jax.ShapeDtypeStruct((2 * nr, 16), jnp.float32),
        mesh=_vmesh(),
        scratch_types=[
            pltpu.VMEM((8, 128), jnp.int32),
            pltpu.VMEM((128, 16), jnp.float32),
            pltpu.VMEM_SHARED((nr, 16), jnp.float32),
            pltpu.SemaphoreType.DMA,
        ],
        compiler_params=_SC_PARAMS,
    )
    def k(idx_hbm, zeros_hbm, out_hbm, idx_v, ones_v, acc, sem):
        c = lax.axis_index("c")
        t = lax.axis_index("s")
        pltpu.sync_copy(zeros_hbm.at[pl.ds(t * rpt, rpt)],
                        acc.at[pl.ds(t * rpt, rpt)])

        @pl.loop(0, 128)
        def _(i):
            ones_v.at[i][...] = jnp.full((16,), 1.0, jnp.float32)

        plsc.subcore_barrier()
        base = (c * per_sc + t * per_tile) // 128

        @pl.loop(0, ngrp)
        def _(g):
            pltpu.sync_copy(idx_hbm.at[pl.ds(base + g * 8, 8)], idx_v)
            for j in range(8):
                pltpu.sync_copy(ones_v, acc.at[idx_v.at[j]], add=True)

        plsc.subcore_barrier()
        pltpu.sync_copy(acc.at[pl.ds(t * rpt, rpt)],
                        out_hbm.at[pl.ds(c * nr + t * rpt, rpt)])

    return k(idx2d, zeros)


def _sc_gather_scatter(table, src2d, gidx2d, nr, d):
    nsrc = table.shape[0] // 2
    etot = src2d.shape[0] * 128
    per_tile = etot // _NS
    ngrp = per_tile // 1024
    rpt = nr // _NS
    zeros = jnp.zeros((nr, d), jnp.float32)

    @functools.partial(
        pl.kernel,
        out_type=jax.ShapeDtypeStruct((2 * nr, d), jnp.float32),
        mesh=_vmesh(),
        scratch_types=[
            pltpu.VMEM((8, 128), jnp.int32),
            pltpu.VMEM((8, 128), jnp.int32),
            pltpu.VMEM((128, d), jnp.float32),
            pltpu.VMEM_SHARED((nr, d), jnp.float32),
            pltpu.SemaphoreType.DMA,
        ],
        compiler_params=_SC_PARAMS,
    )
    def k(tab_hbm, src_hbm, gidx_hbm, zeros_hbm, out_hbm,
          src_v, dst_v, rows_v, acc, sem):
        c = lax.axis_index("c")
        t = lax.axis_index("s")
        pltpu.sync_copy(zeros_hbm.at[pl.ds(t * rpt, rpt)],
                        acc.at[pl.ds(t * rpt, rpt)])
        plsc.subcore_barrier()
        base = t * per_tile // 128
        off = c * nsrc

        @pl.loop(0, ngrp)
        def _(g):
            pltpu.sync_copy(src_hbm.at[pl.ds(base + g * 8, 8)], src_v)
            pltpu.sync_copy(gidx_hbm.at[pl.ds(base + g * 8, 8)], dst_v)
            for j in range(8):
                for kk in range(8):
                    sl = (pl.ds(kk * 16, 16),)
                    src_v.at[j][sl] = src_v.at[j][sl] + off
            for j in range(8):
                pltpu.async_copy(tab_hbm.at[src_v.at[j]], rows_v, sem).wait()
                pltpu.sync_copy(rows_v, acc.at[dst_v.at[j]], add=True)

        plsc.subcore_barrier()
        pltpu.sync_copy(acc.at[pl.ds(t * rpt, rpt)],
                        out_hbm.at[pl.ds(c * nr + t * rpt, rpt)])

    return k(table, src2d, gidx2d, zeros)


def _pad_idx(a, e_pad, fill):
    return jnp.concatenate(
        [a, jnp.full((e_pad - a.shape[0],), fill, jnp.int32)]).reshape(-1, 128)


def _faconv(x, x0, edge_index, w_l, w_r):
    src, dst = edge_index[0], edge_index[1]
    n = x.shape[0]
    deg = jnp.zeros((n,), x.dtype).at[dst].add(jnp.ones(dst.shape, x.dtype))
    dis = jnp.where(deg > 0, lax.rsqrt(jnp.maximum(deg, 1.0)), 0.0)
    ew = dis[src] * dis[dst]
    alpha = jnp.tanh((x @ w_l)[src] + (x @ w_r)[dst])
    out = jnp.zeros_like(x).at[dst].add(x[src] * (alpha * ew)[:, None])
    return out + _EPS_FA * x0


_EP_RGCN = 327680
_NR_RGCN = 20096


def kernel(num_prop_burst, cat_prop_burst, tweet_range_list, edge_index_burst,
           re_index, des, tweet, num_prop, cat_prop, edge_index_rgcn,
           edge_type, params):
    p = params
    act = jax.nn.leaky_relu

    num = act(num_prop_burst @ p['w_num'] + p['b_num'])
    cat = act(cat_prop_burst @ p['w_cat'] + p['b_cat'])
    x = jnp.concatenate([num, cat], axis=1)
    x = act(x @ p['w_tog'] + p['b_tog'])

    x1 = _faconv(x, x, edge_index_burst, p['w_att_l'], p['w_att_r'])
    x2 = _faconv(x1, x, edge_index_burst, p['w_att_l'], p['w_att_r'])
    x2 = (x2 ** 2 + 1e-08) ** 0.5

    num0 = x2.shape[1]
    nseg = tweet_range_list.shape[0] - 1
    pos = jnp.arange(x2.shape[0], dtype=tweet_range_list.dtype)
    seg = jnp.searchsorted(tweet_range_list, pos, side='right') - 1
    seg = jnp.where((seg >= 0) & (seg < nseg), seg, nseg)
    x3 = jax.ops.segment_sum(x2, seg, num_segments=nseg + 1)[:nseg]
    num_users = des.shape[0]
    x3 = jnp.concatenate(
        [x3, jnp.zeros((num_users - nseg, num0), x3.dtype)], axis=0)

    x3 = _sc_gather_rows(x3, re_index)
    x_burst = act(x3 @ p['w_map'] + p['b_map'])

    d = act(des @ p['w_des'] + p['b_des'])
    t = act(tweet @ p['w_tw'] + p['b_tw'])
    n = act(num_prop @ p['w_np'] + p['b_np'])
    c = act(cat_prop @ p['w_cp'] + p['b_cp'])
    xr = jnp.concatenate([d, t, n, c], axis=1)
    xr = act(xr @ p['w_in'] + p['b_in'])

    src_r = _pad_idx(edge_index_rgcn[0], _EP_RGCN, 0)
    gidx_r = _pad_idx(edge_index_rgcn[1] + edge_type * 10000, _EP_RGCN, 20000)
    cnt2 = _sc_count(gidx_r, _NR_RGCN)
    cnt = cnt2[:_NR_RGCN, 0] + cnt2[_NR_RGCN:, 0]
    c0 = jnp.maximum(cnt[:10000], 1.0)
    c1 = jnp.maximum(cnt[10000:20000], 1.0)

    def rgcn_layer(xin):
        xstk = jnp.concatenate([xin[:, :64], xin[:, 64:]], axis=0)
        s = _sc_gather_scatter(xstk, src_r, gidx_r, _NR_RGCN, 64)
        s_a, s_b = s[:_NR_RGCN], s[_NR_RGCN:]
        s0 = jnp.concatenate([s_a[:10000], s_b[:10000]], axis=1)
        s1 = jnp.concatenate([s_a[10000:20000], s_b[10000:20000]], axis=1)
        return (xin @ p['w_root'] + p['b_rgcn']
                + (s0 / c0[:, None]) @ p['w_rgcn'][0]
                + (s1 / c1[:, None]) @ p['w_rgcn'][1])

    xr = rgcn_layer(xr)
    xr = rgcn_layer(xr)
    x_rgcn = act(xr @ p['w_out1'] + p['b_out1'])

    xcat = jnp.concatenate([x_burst, x_rgcn], axis=1)
    xcat = act(xcat @ p['w_f0'] + p['b_f0'])
    return xcat @ p['w_f'] + p['b_f']


# speedup vs baseline: 8.7127x; 8.6100x over previous
import functools

import jax
import jax.numpy as jnp
from jax import lax
from jax.experimental import pallas as pl
from jax.experimental.pallas import tpu as pltpu
from jax.experimental.pallas import tpu_sc as plsc

_EPS_FA = 0.1

_NC = 2
_NS = 16
_NW = _NC * _NS


_SC_PARAMS = pltpu.CompilerParams(
    use_tc_tiling_on_sc=False, needs_layout_passes=False)


def _vmesh():
    return plsc.VectorSubcoreMesh(
        core_axis_name="c", subcore_axis_name="s",
        num_cores=_NC, num_subcores=_NS)


def _wid():
    return lax.axis_index("s") * _NC + lax.axis_index("c")


def _sc_gather_rows(table, idx, *, chunk=80):
    n, d = table.shape
    b = idx.shape[0]
    per_w = -(-b // (_NW * chunk)) * chunk
    bp = per_w * _NW
    nchunk = per_w // chunk
    idx_p = jnp.concatenate([idx, jnp.zeros((bp - b,), jnp.int32)])
    idx2 = idx_p.reshape(_NW * nchunk, chunk)

    @functools.partial(
        pl.kernel,
        out_type=jax.ShapeDtypeStruct((bp, d), jnp.float32),
        mesh=_vmesh(),
        scratch_types=[
            pltpu.VMEM((nchunk, chunk), jnp.int32),
            pltpu.VMEM((chunk, d), jnp.float32),
            pltpu.SemaphoreType.DMA,
        ],
        compiler_params=_SC_PARAMS,
    )
    def k(table_hbm, idx_hbm, out_hbm, idx_v, rows_v, sem):
        w = _wid()
        pltpu.sync_copy(idx_hbm.at[pl.ds(w * nchunk, nchunk)], idx_v)

        @pl.loop(0, nchunk)
        def _(j):
            pltpu.async_copy(table_hbm.at[idx_v.at[j]], rows_v, sem).wait()
            pltpu.sync_copy(
                rows_v, out_hbm.at[pl.ds(w * per_w + j * chunk, chunk)])

    return k(table, idx2)[:b]


def _sc_count(idx2d, nr):
    etot = idx2d.shape[0] * 128
    per_sc = etot // 2
    per_tile = per_sc // _NS
    ngrp = per_tile // 1024
    rpt = nr // _NS
    zeros = jnp.zeros((nr, 16), jnp.float32)

    @functools.partial(
        pl.kernel,
        out_type=jax.ShapeDtypeStruct((2 * nr, 16), jnp.float32),
        mesh=_vmesh(),
        scratch_types=[
            pltpu.VMEM((8, 128), jnp.int32),
            pltpu.VMEM((128, 16), jnp.float32),
            pltpu.VMEM_SHARED((nr, 16), jnp.float32),
            pltpu.SemaphoreType.DMA,
        ],
        compiler_params=_SC_PARAMS,
    )
    def k(idx_hbm, zeros_hbm, out_hbm, idx_v, ones_v, acc, sem):
        c = lax.axis_index("c")
        t = lax.axis_index("s")
        pltpu.sync_copy(zeros_hbm.at[pl.ds(t * rpt, rpt)],
                        acc.at[pl.ds(t * rpt, rpt)])

        @pl.loop(0, 128)
        def _(i):
            ones_v.at[i][...] = jnp.full((16,), 1.0, jnp.float32)

        plsc.subcore_barrier()
        base = (c * per_sc + t * per_tile) // 128

        @pl.loop(0, ngrp)
        def _(g):
            pltpu.sync_copy(idx_hbm.at[pl.ds(base + g * 8, 8)], idx_v)
            for j in range(8):
                pltpu.sync_copy(ones_v, acc.at[idx_v.at[j]], add=True)

        plsc.subcore_barrier()
        pltpu.sync_copy(acc.at[pl.ds(t * rpt, rpt)],
                        out_hbm.at[pl.ds(c * nr + t * rpt, rpt)])

    return k(idx2d, zeros)


def _sc_gather_scatter(table, src2d, gidx2d, nr, d):
    nsrc = table.shape[0] // 2
    etot = src2d.shape[0] * 128
    per_tile = etot // _NS
    ngrp = per_tile // 1024
    rpt = nr // _NS
    zeros = jnp.zeros((nr, d), jnp.float32)

    @functools.partial(
        pl.kernel,
        out_type=jax.ShapeDtypeStruct((2 * nr, d), jnp.float32),
        mesh=_vmesh(),
        scratch_types=[
            pltpu.VMEM((8, 128), jnp.int32),
            pltpu.VMEM((8, 128), jnp.int32),
            pltpu.VMEM((128, d), jnp.float32),
            pltpu.VMEM_SHARED((nr, d), jnp.float32),
            pltpu.SemaphoreType.DMA,
        ],
        compiler_params=_SC_PARAMS,
    )
    def k(tab_hbm, src_hbm, gidx_hbm, zeros_hbm, out_hbm,
          src_v, dst_v, rows_v, acc, sem):
        c = lax.axis_index("c")
        t = lax.axis_index("s")
        pltpu.sync_copy(zeros_hbm.at[pl.ds(t * rpt, rpt)],
                        acc.at[pl.ds(t * rpt, rpt)])
        plsc.subcore_barrier()
        base = t * per_tile // 128
        off = c * nsrc

        @pl.loop(0, ngrp)
        def _(g):
            pltpu.sync_copy(src_hbm.at[pl.ds(base + g * 8, 8)], src_v)
            pltpu.sync_copy(gidx_hbm.at[pl.ds(base + g * 8, 8)], dst_v)
            for j in range(8):
                for kk in range(8):
                    sl = (pl.ds(kk * 16, 16),)
                    src_v.at[j][sl] = src_v.at[j][sl] + off
            for j in range(8):
                pltpu.async_copy(tab_hbm.at[src_v.at[j]], rows_v, sem).wait()
                pltpu.sync_copy(rows_v, acc.at[dst_v.at[j]], add=True)

        plsc.subcore_barrier()
        pltpu.sync_copy(acc.at[pl.ds(t * rpt, rpt)],
                        out_hbm.at[pl.ds(c * nr + t * rpt, rpt)])

    return k(table, src2d, gidx2d, zeros)


def _sc_edge_map(table, streams, fn):
    erows = streams[0].shape[0]
    per_tile = erows * 128 // _NW
    ngrp = per_tile // 1024
    ntab = table.shape[0]

    @functools.partial(
        pl.kernel,
        out_type=jax.ShapeDtypeStruct((erows, 128), jnp.float32),
        mesh=_vmesh(),
        scratch_types=(
            [pltpu.VMEM((ntab,), jnp.float32)]
            + [pltpu.VMEM((8, 128), s.dtype) for s in streams]
            + [pltpu.VMEM((8, 128), jnp.float32)]
        ),
        compiler_params=_SC_PARAMS,
    )
    def k(tab_hbm, *rest):
        stream_hbms = rest[:len(streams)]
        out_hbm = rest[len(streams)]
        tab_v = rest[len(streams) + 1]
        stream_vs = rest[len(streams) + 2:-1]
        out_v = rest[-1]
        w = _wid()
        pltpu.sync_copy(tab_hbm, tab_v)
        base = w * per_tile // 128

        @pl.loop(0, ngrp)
        def _(g):
            for sh, sv in zip(stream_hbms, stream_vs):
                pltpu.sync_copy(sh.at[pl.ds(base + g * 8, 8)], sv)
            for j in range(8):
                for kk in range(8):
                    sl = pl.ds(kk * 16, 16)
                    vecs = [sv.at[j][sl] for sv in stream_vs]
                    out_v.at[j][sl] = fn(tab_v, vecs)
            pltpu.sync_copy(out_v, out_hbm.at[pl.ds(base + g * 8, 8)])

    return k(table, *streams)


def _lane_bcast(v, q):
    idx = jnp.full((16, 1), q, jnp.int32)
    dn = lax.GatherDimensionNumbers(
        offset_dims=(), collapsed_slice_dims=(0,), start_index_map=(0,))
    return lax.gather(v, idx, dn, (1,),
                      mode=lax.GatherScatterMode.PROMISE_IN_BOUNDS)


def _sc_gather_scale_scatter(table, src2d, dst2d, w2d, nr):
    nsrc = table.shape[0] // 2
    etot = src2d.shape[0] * 128
    per_tile = etot // _NS
    ngrp = per_tile // 1024
    rpt = nr // _NS
    zeros = jnp.zeros((nr, 16), jnp.float32)

    @functools.partial(
        pl.kernel,
        out_type=jax.ShapeDtypeStruct((2 * nr, 16), jnp.float32),
        mesh=_vmesh(),
        scratch_types=[
            pltpu.VMEM((8, 128), jnp.int32),
            pltpu.VMEM((8, 128), jnp.int32),
            pltpu.VMEM((8, 128), jnp.float32),
            pltpu.VMEM((128, 16), jnp.float32),
            pltpu.VMEM_SHARED((nr, 16), jnp.float32),
            pltpu.SemaphoreType.DMA,
        ],
        compiler_params=_SC_PARAMS,
    )
    def k(tab_hbm, src_hbm, dst_hbm, w_hbm, zeros_hbm, out_hbm,
          src_v, dst_v, w_v, rows_v, acc, sem):
        c = lax.axis_index("c")
        t = lax.axis_index("s")
        pltpu.sync_copy(zeros_hbm.at[pl.ds(t * rpt, rpt)],
                        acc.at[pl.ds(t * rpt, rpt)])
        plsc.subcore_barrier()
        base = t * per_tile // 128
        off = c * nsrc

        @pl.loop(0, ngrp)
        def _(g):
            pltpu.sync_copy(src_hbm.at[pl.ds(base + g * 8, 8)], src_v)
            pltpu.sync_copy(dst_hbm.at[pl.ds(base + g * 8, 8)], dst_v)
            pltpu.sync_copy(w_hbm.at[pl.ds(base + g * 8, 8)], w_v)

            @pl.loop(0, 8)
            def _(j):
                for kk in range(8):
                    sl = pl.ds(kk * 16, 16)
                    src_v.at[j][sl] = src_v.at[j][sl] + off
                pltpu.async_copy(tab_hbm.at[src_v.at[j]], rows_v, sem).wait()
                for e16 in range(8):
                    wv = w_v.at[j][pl.ds(e16 * 16, 16)]
                    for q in range(16):
                        e = e16 * 16 + q
                        rows_v.at[e][...] = (
                            rows_v.at[e][...] * _lane_bcast(wv, q))
                pltpu.sync_copy(rows_v, acc.at[dst_v.at[j]], add=True)

        plsc.subcore_barrier()
        pltpu.sync_copy(acc.at[pl.ds(t * rpt, rpt)],
                        out_hbm.at[pl.ds(c * nr + t * rpt, rpt)])

    return k(table, src2d, dst2d, w2d, zeros)


def _pad_idx(a, e_pad, fill):
    return jnp.concatenate(
        [a, jnp.full((e_pad - a.shape[0],), fill, jnp.int32)]).reshape(-1, 128)


_EP_FA = 1605632
_NP_FA = 100096
_EP_RGCN = 327680
_NR_RGCN = 20096


def kernel(num_prop_burst, cat_prop_burst, tweet_range_list, edge_index_burst,
           re_index, des, tweet, num_prop, cat_prop, edge_index_rgcn,
           edge_type, params):
    p = params
    act = jax.nn.leaky_relu

    num = act(num_prop_burst @ p['w_num'] + p['b_num'])
    cat = act(cat_prop_burst @ p['w_cat'] + p['b_cat'])
    x = jnp.concatenate([num, cat], axis=1)
    x = act(x @ p['w_tog'] + p['b_tog'])

    src_b = _pad_idx(edge_index_burst[0], _EP_FA, 0)
    dst_b = _pad_idx(edge_index_burst[1], _EP_FA, 100000)
    deg2 = _sc_count(dst_b, _NP_FA)
    deg = deg2[:_NP_FA, 0] + deg2[_NP_FA:, 0]
    dis = jnp.where(deg > 0, lax.rsqrt(jnp.maximum(deg, 1.0)), 0.0)
    dis_e = _sc_edge_map(
        dis, [src_b, dst_b],
        lambda tab, v: (plsc.load_gather(tab, [v[0]])
                        * plsc.load_gather(tab, [v[1]])))

    def faconv_layer(xin, x0):
        l_pad = jnp.concatenate(
            [xin @ p['w_att_l'], jnp.zeros((_NP_FA - 100000,), jnp.float32)])
        r_pad = jnp.concatenate(
            [xin @ p['w_att_r'], jnp.zeros((_NP_FA - 100000,), jnp.float32)])
        ls = _sc_edge_map(l_pad, [src_b],
                          lambda tab, v: plsc.load_gather(tab, [v[0]]))

        def wfn(tab, v):
            z = v[1] + plsc.load_gather(tab, [v[0]])
            return (1.0 - 2.0 / (jnp.exp(2.0 * z) + 1.0)) * v[2]

        w2d = _sc_edge_map(r_pad, [dst_b, ls, dis_e], wfn)
        xstk = jnp.concatenate([
            jnp.pad(xin[:, :16], ((0, _NP_FA - 100000), (0, 0))),
            jnp.pad(xin[:, 16:], ((0, _NP_FA - 100000), (0, 0)))], axis=0)
        out = _sc_gather_scale_scatter(xstk, src_b, dst_b, w2d, _NP_FA)
        o = jnp.concatenate(
            [out[:100000], out[_NP_FA:_NP_FA + 100000]], axis=1)
        return o + _EPS_FA * x0

    x1 = faconv_layer(x, x)
    x2 = faconv_layer(x1, x)
    x2 = (x2 ** 2 + 1e-08) ** 0.5

    num0 = x2.shape[1]
    nseg = tweet_range_list.shape[0] - 1
    pos = jnp.arange(x2.shape[0], dtype=tweet_range_list.dtype)
    seg = jnp.searchsorted(tweet_range_list, pos, side='right') - 1
    seg = jnp.where((seg >= 0) & (seg < nseg), seg, nseg)
    x3 = jax.ops.segment_sum(x2, seg, num_segments=nseg + 1)[:nseg]
    num_users = des.shape[0]
    x3 = jnp.concatenate(
        [x3, jnp.zeros((num_users - nseg, num0), x3.dtype)], axis=0)

    x3 = _sc_gather_rows(x3, re_index)
    x_burst = act(x3 @ p['w_map'] + p['b_map'])

    d = act(des @ p['w_des'] + p['b_des'])
    t = act(tweet @ p['w_tw'] + p['b_tw'])
    n = act(num_prop @ p['w_np'] + p['b_np'])
    c = act(cat_prop @ p['w_cp'] + p['b_cp'])
    xr = jnp.concatenate([d, t, n, c], axis=1)
    xr = act(xr @ p['w_in'] + p['b_in'])

    src_r = _pad_idx(edge_index_rgcn[0], _EP_RGCN, 0)
    gidx_r = _pad_idx(edge_index_rgcn[1] + edge_type * 10000, _EP_RGCN, 20000)
    cnt2 = _sc_count(gidx_r, _NR_RGCN)
    cnt = cnt2[:_NR_RGCN, 0] + cnt2[_NR_RGCN:, 0]
    c0 = jnp.maximum(cnt[:10000], 1.0)
    c1 = jnp.maximum(cnt[10000:20000], 1.0)

    def rgcn_layer(xin):
        xstk = jnp.concatenate([xin[:, :64], xin[:, 64:]], axis=0)
        s = _sc_gather_scatter(xstk, src_r, gidx_r, _NR_RGCN, 64)
        s_a, s_b = s[:_NR_RGCN], s[_NR_RGCN:]
        s0 = jnp.concatenate([s_a[:10000], s_b[:10000]], axis=1)
        s1 = jnp.concatenate([s_a[10000:20000], s_b[10000:20000]], axis=1)
        return (xin @ p['w_root'] + p['b_rgcn']
                + (s0 / c0[:, None]) @ p['w_rgcn'][0]
                + (s1 / c1[:, None]) @ p['w_rgcn'][1])

    xr = rgcn_layer(xr)
    xr = rgcn_layer(xr)
    x_rgcn = act(xr @ p['w_out1'] + p['b_out1'])

    xcat = jnp.concatenate([x_burst, x_rgcn], axis=1)
    xcat = act(xcat @ p['w_f0'] + p['b_f0'])
    return xcat @ p['w_f'] + p['b_f']


# speedup vs baseline: 20.4831x; 2.3509x over previous
import functools

import jax
import jax.numpy as jnp
from jax import lax
from jax.experimental import pallas as pl
from jax.experimental.pallas import tpu as pltpu
from jax.experimental.pallas import tpu_sc as plsc

_EPS_FA = 0.1

_NC = 2
_NS = 16
_NW = _NC * _NS


_SC_PARAMS = pltpu.CompilerParams(
    use_tc_tiling_on_sc=False, needs_layout_passes=False)


def _vmesh():
    return plsc.VectorSubcoreMesh(
        core_axis_name="c", subcore_axis_name="s",
        num_cores=_NC, num_subcores=_NS)


def _wid():
    return lax.axis_index("s") * _NC + lax.axis_index("c")


def _sc_gather_rows(table, idx, *, chunk=80):
    n, d = table.shape
    b = idx.shape[0]
    per_w = -(-b // (_NW * chunk)) * chunk
    bp = per_w * _NW
    nchunk = per_w // chunk
    idx_p = jnp.concatenate([idx, jnp.zeros((bp - b,), jnp.int32)])
    idx2 = idx_p.reshape(_NW * nchunk, chunk)

    @functools.partial(
        pl.kernel,
        out_type=jax.ShapeDtypeStruct((bp, d), jnp.float32),
        mesh=_vmesh(),
        scratch_types=[
            pltpu.VMEM((nchunk, chunk), jnp.int32),
            pltpu.VMEM((chunk, d), jnp.float32),
            pltpu.SemaphoreType.DMA,
        ],
        compiler_params=_SC_PARAMS,
    )
    def k(table_hbm, idx_hbm, out_hbm, idx_v, rows_v, sem):
        w = _wid()
        pltpu.sync_copy(idx_hbm.at[pl.ds(w * nchunk, nchunk)], idx_v)

        @pl.loop(0, nchunk)
        def _(j):
            pltpu.async_copy(table_hbm.at[idx_v.at[j]], rows_v, sem).wait()
            pltpu.sync_copy(
                rows_v, out_hbm.at[pl.ds(w * per_w + j * chunk, chunk)])

    return k(table, idx2)[:b]


def _sc_count(idx2d, nr):
    etot = idx2d.shape[0] * 128
    per_sc = etot // 2
    per_tile = per_sc // _NS
    ngrp = per_tile // 1024
    rpt = nr // _NS
    zeros = jnp.zeros((nr, 16), jnp.float32)

    @functools.partial(
        pl.kernel,
        out_type=jax.ShapeDtypeStruct((2 * nr, 16), jnp.float32),
        mesh=_vmesh(),
        scratch_types=[
            pltpu.VMEM((8, 128), jnp.int32),
            pltpu.VMEM((128, 16), jnp.float32),
            pltpu.VMEM_SHARED((nr, 16), jnp.float32),
            pltpu.SemaphoreType.DMA,
        ],
        compiler_params=_SC_PARAMS,
    )
    def k(idx_hbm, zeros_hbm, out_hbm, idx_v, ones_v, acc, sem):
        c = lax.axis_index("c")
        t = lax.axis_index("s")
        pltpu.sync_copy(zeros_hbm.at[pl.ds(t * rpt, rpt)],
                        acc.at[pl.ds(t * rpt, rpt)])

        @pl.loop(0, 128)
        def _(i):
            ones_v.at[i][...] = jnp.full((16,), 1.0, jnp.float32)

        plsc.subcore_barrier()
        base = (c * per_sc + t * per_tile) // 128

        @pl.loop(0, ngrp)
        def _(g):
            pltpu.sync_copy(idx_hbm.at[pl.ds(base + g * 8, 8)], idx_v)
            for j in range(8):
                pltpu.sync_copy(ones_v, acc.at[idx_v.at[j]], add=True)

        plsc.subcore_barrier()
        pltpu.sync_copy(acc.at[pl.ds(t * rpt, rpt)],
                        out_hbm.at[pl.ds(c * nr + t * rpt, rpt)])

    return k(idx2d, zeros)


def _sc_gather_scatter(table, src2d, gidx2d, nr, d):
    nsrc = table.shape[0] // 2
    etot = src2d.shape[0] * 128
    per_tile = etot // _NS
    ngrp = per_tile // 1024
    rpt = nr // _NS
    zeros = jnp.zeros((nr, d), jnp.float32)

    @functools.partial(
        pl.kernel,
        out_type=jax.ShapeDtypeStruct((2 * nr, d), jnp.float32),
        mesh=_vmesh(),
        scratch_types=[
            pltpu.VMEM((8, 128), jnp.int32),
            pltpu.VMEM((8, 128), jnp.int32),
            pltpu.VMEM((128, d), jnp.float32),
            pltpu.VMEM_SHARED((nr, d), jnp.float32),
            pltpu.SemaphoreType.DMA,
        ],
        compiler_params=_SC_PARAMS,
    )
    def k(tab_hbm, src_hbm, gidx_hbm, zeros_hbm, out_hbm,
          src_v, dst_v, rows_v, acc, sem):
        c = lax.axis_index("c")
        t = lax.axis_index("s")
        pltpu.sync_copy(zeros_hbm.at[pl.ds(t * rpt, rpt)],
                        acc.at[pl.ds(t * rpt, rpt)])
        plsc.subcore_barrier()
        base = t * per_tile // 128
        off = c * nsrc

        @pl.loop(0, ngrp)
        def _(g):
            pltpu.sync_copy(src_hbm.at[pl.ds(base + g * 8, 8)], src_v)
            pltpu.sync_copy(gidx_hbm.at[pl.ds(base + g * 8, 8)], dst_v)
            for j in range(8):
                for kk in range(8):
                    sl = (pl.ds(kk * 16, 16),)
                    src_v.at[j][sl] = src_v.at[j][sl] + off
            for j in range(8):
                pltpu.async_copy(tab_hbm.at[src_v.at[j]], rows_v, sem).wait()
                pltpu.sync_copy(rows_v, acc.at[dst_v.at[j]], add=True)

        plsc.subcore_barrier()
        pltpu.sync_copy(acc.at[pl.ds(t * rpt, rpt)],
                        out_hbm.at[pl.ds(c * nr + t * rpt, rpt)])

    return k(table, src2d, gidx2d, zeros)


def _sc_edge_map(table, streams, fn):
    erows = streams[0].shape[0]
    per_tile = erows * 128 // _NW
    ngrp = per_tile // 1024
    ntab = table.shape[0]

    @functools.partial(
        pl.kernel,
        out_type=jax.ShapeDtypeStruct((erows, 128), jnp.float32),
        mesh=_vmesh(),
        scratch_types=(
            [pltpu.VMEM((ntab,), jnp.float32)]
            + [pltpu.VMEM((8, 128), s.dtype) for s in streams]
            + [pltpu.VMEM((8, 128), jnp.float32)]
        ),
        compiler_params=_SC_PARAMS,
    )
    def k(tab_hbm, *rest):
        stream_hbms = rest[:len(streams)]
        out_hbm = rest[len(streams)]
        tab_v = rest[len(streams) + 1]
        stream_vs = rest[len(streams) + 2:-1]
        out_v = rest[-1]
        w = _wid()
        pltpu.sync_copy(tab_hbm, tab_v)
        base = w * per_tile // 128

        @pl.loop(0, ngrp)
        def _(g):
            for sh, sv in zip(stream_hbms, stream_vs):
                pltpu.sync_copy(sh.at[pl.ds(base + g * 8, 8)], sv)
            for j in range(8):
                for kk in range(8):
                    sl = pl.ds(kk * 16, 16)
                    vecs = [sv.at[j][sl] for sv in stream_vs]
                    out_v.at[j][sl] = fn(tab_v, vecs)
            pltpu.sync_copy(out_v, out_hbm.at[pl.ds(base + g * 8, 8)])

    return k(table, *streams)


def _lane_bcast(v, q):
    idx = jnp.full((16, 1), q, jnp.int32)
    dn = lax.GatherDimensionNumbers(
        offset_dims=(), collapsed_slice_dims=(0,), start_index_map=(0,))
    return lax.gather(v, idx, dn, (1,),
                      mode=lax.GatherScatterMode.PROMISE_IN_BOUNDS)


def _sc_gather_scale_scatter(table, src2d, dst2d, w2d, nr):
    nsrc = table.shape[0] // 2
    etot = src2d.shape[0] * 128
    per_tile = etot // _NS
    ngrp = per_tile // 1024
    rpt = nr // _NS
    zeros = jnp.zeros((nr, 16), jnp.float32)

    @functools.partial(
        pl.kernel,
        out_type=jax.ShapeDtypeStruct((2 * nr, 16), jnp.float32),
        mesh=_vmesh(),
        scratch_types=[
            pltpu.VMEM((8, 128), jnp.int32),
            pltpu.VMEM((8, 128), jnp.int32),
            pltpu.VMEM((8, 128), jnp.float32),
            pltpu.VMEM((128, 16), jnp.float32),
            pltpu.VMEM_SHARED((nr, 16), jnp.float32),
            pltpu.SemaphoreType.DMA,
        ],
        compiler_params=_SC_PARAMS,
    )
    def k(tab_hbm, src_hbm, dst_hbm, w_hbm, zeros_hbm, out_hbm,
          src_v, dst_v, w_v, rows_v, acc, sem):
        c = lax.axis_index("c")
        t = lax.axis_index("s")
        pltpu.sync_copy(zeros_hbm.at[pl.ds(t * rpt, rpt)],
                        acc.at[pl.ds(t * rpt, rpt)])
        plsc.subcore_barrier()
        base = t * per_tile // 128
        off = c * nsrc

        @pl.loop(0, ngrp)
        def _(g):
            pltpu.sync_copy(src_hbm.at[pl.ds(base + g * 8, 8)], src_v)
            pltpu.sync_copy(dst_hbm.at[pl.ds(base + g * 8, 8)], dst_v)
            pltpu.sync_copy(w_hbm.at[pl.ds(base + g * 8, 8)], w_v)

            @pl.loop(0, 8)
            def _(j):
                for kk in range(8):
                    sl = pl.ds(kk * 16, 16)
                    src_v.at[j][sl] = src_v.at[j][sl] + off
                pltpu.async_copy(tab_hbm.at[src_v.at[j]], rows_v, sem).wait()
                for e16 in range(8):
                    wv = w_v.at[j][pl.ds(e16 * 16, 16)]
                    for q in range(16):
                        e = e16 * 16 + q
                        rows_v.at[e][...] = (
                            rows_v.at[e][...] * _lane_bcast(wv, q))
                pltpu.sync_copy(rows_v, acc.at[dst_v.at[j]], add=True)

        plsc.subcore_barrier()
        pltpu.sync_copy(acc.at[pl.ds(t * rpt, rpt)],
                        out_hbm.at[pl.ds(c * nr + t * rpt, rpt)])

    return k(table, src2d, dst2d, w2d, zeros)


_NH_SEG = 102400
_NS_SEG = 10112


def _sc_segsum(x2stk, seg2d):
    per_tile = _NH_SEG // _NS
    rpt = _NS_SEG // _NS
    zeros = jnp.zeros((_NS_SEG, 16), jnp.float32)

    @functools.partial(
        pl.kernel,
        out_type=jax.ShapeDtypeStruct((2 * _NS_SEG, 16), jnp.float32),
        mesh=_vmesh(),
        scratch_types=[
            pltpu.VMEM((5, 128), jnp.int32),
            pltpu.VMEM((640, 16), jnp.float32),
            pltpu.VMEM_SHARED((_NS_SEG, 16), jnp.float32),
        ],
        compiler_params=_SC_PARAMS,
    )
    def k(x_hbm, seg_hbm, zeros_hbm, out_hbm, seg_v, rows_v, acc):
        c = lax.axis_index("c")
        t = lax.axis_index("s")
        pltpu.sync_copy(zeros_hbm.at[pl.ds(t * rpt, rpt)],
                        acc.at[pl.ds(t * rpt, rpt)])
        plsc.subcore_barrier()
        base_r = c * _NH_SEG + t * per_tile
        base_s = t * (per_tile // 128)

        @pl.loop(0, 10)
        def _(g):
            pltpu.sync_copy(x_hbm.at[pl.ds(base_r + g * 640, 640)], rows_v)
            pltpu.sync_copy(seg_hbm.at[pl.ds(base_s + g * 5, 5)], seg_v)
            for j in range(5):
                pltpu.sync_copy(rows_v.at[pl.ds(j * 128, 128)],
                                acc.at[seg_v.at[j]], add=True)

        plsc.subcore_barrier()
        pltpu.sync_copy(acc.at[pl.ds(t * rpt, rpt)],
                        out_hbm.at[pl.ds(c * _NS_SEG + t * rpt, rpt)])

    return k(x2stk, seg2d, zeros)


def _pad_idx(a, e_pad, fill):
    return jnp.concatenate(
        [a, jnp.full((e_pad - a.shape[0],), fill, jnp.int32)]).reshape(-1, 128)


_EP_FA = 1605632
_NP_FA = 100096
_EP_RGCN = 327680
_NR_RGCN = 20096


def kernel(num_prop_burst, cat_prop_burst, tweet_range_list, edge_index_burst,
           re_index, des, tweet, num_prop, cat_prop, edge_index_rgcn,
           edge_type, params):
    p = params
    act = jax.nn.leaky_relu

    num = act(num_prop_burst @ p['w_num'] + p['b_num'])
    cat = act(cat_prop_burst @ p['w_cat'] + p['b_cat'])
    x = jnp.concatenate([num, cat], axis=1)
    x = act(x @ p['w_tog'] + p['b_tog'])

    src_b = _pad_idx(edge_index_burst[0], _EP_FA, 0)
    dst_b = _pad_idx(edge_index_burst[1], _EP_FA, 100000)
    deg2 = _sc_count(dst_b, _NP_FA)
    deg = deg2[:_NP_FA, 0] + deg2[_NP_FA:, 0]
    dis = jnp.where(deg > 0, lax.rsqrt(jnp.maximum(deg, 1.0)), 0.0)
    dis_e = _sc_edge_map(
        dis, [src_b, dst_b],
        lambda tab, v: (plsc.load_gather(tab, [v[0]])
                        * plsc.load_gather(tab, [v[1]])))

    def faconv_layer(xin, x0):
        l_pad = jnp.concatenate(
            [xin @ p['w_att_l'], jnp.zeros((_NP_FA - 100000,), jnp.float32)])
        r_pad = jnp.concatenate(
            [xin @ p['w_att_r'], jnp.zeros((_NP_FA - 100000,), jnp.float32)])
        ls = _sc_edge_map(l_pad, [src_b],
                          lambda tab, v: plsc.load_gather(tab, [v[0]]))

        def wfn(tab, v):
            z = v[1] + plsc.load_gather(tab, [v[0]])
            return (1.0 - 2.0 / (jnp.exp(2.0 * z) + 1.0)) * v[2]

        w2d = _sc_edge_map(r_pad, [dst_b, ls, dis_e], wfn)
        xstk = jnp.concatenate([
            jnp.pad(xin[:, :16], ((0, _NP_FA - 100000), (0, 0))),
            jnp.pad(xin[:, 16:], ((0, _NP_FA - 100000), (0, 0)))], axis=0)
        out = _sc_gather_scale_scatter(xstk, src_b, dst_b, w2d, _NP_FA)
        o = jnp.concatenate(
            [out[:100000], out[_NP_FA:_NP_FA + 100000]], axis=1)
        return o + _EPS_FA * x0

    x1 = faconv_layer(x, x)
    x2 = faconv_layer(x1, x)
    x2 = (x2 ** 2 + 1e-08) ** 0.5

    hist = jnp.zeros((100000,), jnp.int32).at[tweet_range_list].add(1)
    seg = jnp.cumsum(hist) - 1
    seg = jnp.where((seg >= 0) & (seg < 10000), seg, 10000)
    seg2d = jnp.concatenate(
        [seg, jnp.full((_NH_SEG - 100000,), 10000, jnp.int32)]).reshape(-1, 128)
    x2stk = jnp.concatenate([
        jnp.pad(x2[:, :16], ((0, _NH_SEG - 100000), (0, 0))),
        jnp.pad(x2[:, 16:], ((0, _NH_SEG - 100000), (0, 0)))], axis=0)
    segsum = _sc_segsum(x2stk, seg2d)
    x3 = jnp.concatenate(
        [segsum[:10000], segsum[_NS_SEG:_NS_SEG + 10000]], axis=1)

    x3 = _sc_gather_rows(x3, re_index)
    x_burst = act(x3 @ p['w_map'] + p['b_map'])

    d = act(des @ p['w_des'] + p['b_des'])
    t = act(tweet @ p['w_tw'] + p['b_tw'])
    n = act(num_prop @ p['w_np'] + p['b_np'])
    c = act(cat_prop @ p['w_cp'] + p['b_cp'])
    xr = jnp.concatenate([d, t, n, c], axis=1)
    xr = act(xr @ p['w_in'] + p['b_in'])

    src_r = _pad_idx(edge_index_rgcn[0], _EP_RGCN, 0)
    gidx_r = _pad_idx(edge_index_rgcn[1] + edge_type * 10000, _EP_RGCN, 20000)
    cnt2 = _sc_count(gidx_r, _NR_RGCN)
    cnt = cnt2[:_NR_RGCN, 0] + cnt2[_NR_RGCN:, 0]
    c0 = jnp.maximum(cnt[:10000], 1.0)
    c1 = jnp.maximum(cnt[10000:20000], 1.0)

    def rgcn_layer(xin):
        xstk = jnp.concatenate([xin[:, :64], xin[:, 64:]], axis=0)
        s = _sc_gather_scatter(xstk, src_r, gidx_r, _NR_RGCN, 64)
        s_a, s_b = s[:_NR_RGCN], s[_NR_RGCN:]
        s0 = jnp.concatenate([s_a[:10000], s_b[:10000]], axis=1)
        s1 = jnp.concatenate([s_a[10000:20000], s_b[10000:20000]], axis=1)
        return (xin @ p['w_root'] + p['b_rgcn']
                + (s0 / c0[:, None]) @ p['w_rgcn'][0]
                + (s1 / c1[:, None]) @ p['w_rgcn'][1])

    xr = rgcn_layer(xr)
    xr = rgcn_layer(xr)
    x_rgcn = act(xr @ p['w_out1'] + p['b_out1'])

    xcat = jnp.concatenate([x_burst, x_rgcn], axis=1)
    xcat = act(xcat @ p['w_f0'] + p['b_f0'])
    return xcat @ p['w_f'] + p['b_f']


# speedup vs baseline: 21.4501x; 1.0472x over previous
import functools

import jax
import jax.numpy as jnp
from jax import lax
from jax.experimental import pallas as pl
from jax.experimental.pallas import tpu as pltpu
from jax.experimental.pallas import tpu_sc as plsc

_EPS_FA = 0.1

_NC = 2
_NS = 16
_NW = _NC * _NS


_SC_PARAMS = pltpu.CompilerParams(
    use_tc_tiling_on_sc=False, needs_layout_passes=False)


def _vmesh():
    return plsc.VectorSubcoreMesh(
        core_axis_name="c", subcore_axis_name="s",
        num_cores=_NC, num_subcores=_NS)


def _wid():
    return lax.axis_index("s") * _NC + lax.axis_index("c")


def _sc_gather_rows(table, idx, *, chunk=80):
    n, d = table.shape
    b = idx.shape[0]
    per_w = -(-b // (_NW * chunk)) * chunk
    bp = per_w * _NW
    nchunk = per_w // chunk
    idx_p = jnp.concatenate([idx, jnp.zeros((bp - b,), jnp.int32)])
    idx2 = idx_p.reshape(_NW * nchunk, chunk)

    @functools.partial(
        pl.kernel,
        out_type=jax.ShapeDtypeStruct((bp, d), jnp.float32),
        mesh=_vmesh(),
        scratch_types=[
            pltpu.VMEM((nchunk, chunk), jnp.int32),
            pltpu.VMEM((chunk, d), jnp.float32),
            pltpu.SemaphoreType.DMA,
        ],
        compiler_params=_SC_PARAMS,
    )
    def k(table_hbm, idx_hbm, out_hbm, idx_v, rows_v, sem):
        w = _wid()
        pltpu.sync_copy(idx_hbm.at[pl.ds(w * nchunk, nchunk)], idx_v)

        @pl.loop(0, nchunk)
        def _(j):
            pltpu.async_copy(table_hbm.at[idx_v.at[j]], rows_v, sem).wait()
            pltpu.sync_copy(
                rows_v, out_hbm.at[pl.ds(w * per_w + j * chunk, chunk)])

    return k(table, idx2)[:b]


def _sc_count(idx2d, nr):
    etot = idx2d.shape[0] * 128
    per_sc = etot // 2
    per_tile = per_sc // _NS
    ngrp = per_tile // 1024
    rpt = nr // _NS
    zeros = jnp.zeros((nr, 16), jnp.float32)

    @functools.partial(
        pl.kernel,
        out_type=jax.ShapeDtypeStruct((2 * nr, 16), jnp.float32),
        mesh=_vmesh(),
        scratch_types=[
            pltpu.VMEM((8, 128), jnp.int32),
            pltpu.VMEM((128, 16), jnp.float32),
            pltpu.VMEM_SHARED((nr, 16), jnp.float32),
            pltpu.SemaphoreType.DMA,
        ],
        compiler_params=_SC_PARAMS,
    )
    def k(idx_hbm, zeros_hbm, out_hbm, idx_v, ones_v, acc, sem):
        c = lax.axis_index("c")
        t = lax.axis_index("s")
        pltpu.sync_copy(zeros_hbm.at[pl.ds(t * rpt, rpt)],
                        acc.at[pl.ds(t * rpt, rpt)])

        @pl.loop(0, 128)
        def _(i):
            ones_v.at[i][...] = jnp.full((16,), 1.0, jnp.float32)

        plsc.subcore_barrier()
        base = (c * per_sc + t * per_tile) // 128

        @pl.loop(0, ngrp)
        def _(g):
            pltpu.sync_copy(idx_hbm.at[pl.ds(base + g * 8, 8)], idx_v)
            for j in range(8):
                pltpu.sync_copy(ones_v, acc.at[idx_v.at[j]], add=True)

        plsc.subcore_barrier()
        pltpu.sync_copy(acc.at[pl.ds(t * rpt, rpt)],
                        out_hbm.at[pl.ds(c * nr + t * rpt, rpt)])

    return k(idx2d, zeros)


def _sc_msg_pass(table, src2d, dst2d, w2d, nr, d, gsz):
    nsrc = table.shape[0] // 2
    etot = src2d.shape[0] * 128
    per_tile = etot // _NS
    kk = gsz // 128
    ngrp = per_tile // gsz
    assert ngrp % 2 == 0
    rpt = nr // _NS
    zeros = jnp.zeros((nr, d), jnp.float32)
    has_w = w2d is not None
    ninp = 5 if has_w else 4

    idx_bufs = [pltpu.VMEM((kk, 128), jnp.int32) for _ in range(4)]
    w_bufs = [pltpu.VMEM((kk, 128), jnp.float32) for _ in range(2)] \
        if has_w else []
    row_bufs = [pltpu.VMEM((gsz, d), jnp.float32) for _ in range(2)]
    sems = [pltpu.SemaphoreType.DMA for _ in range(6)]

    @functools.partial(
        pl.kernel,
        out_type=jax.ShapeDtypeStruct((2 * nr, d), jnp.float32),
        mesh=_vmesh(),
        scratch_types=(idx_bufs + w_bufs + row_bufs
                       + [pltpu.VMEM_SHARED((nr, d), jnp.float32)] + sems),
        compiler_params=_SC_PARAMS,
    )
    def k(*refs):
        tab_hbm, src_hbm, dst_hbm = refs[0], refs[1], refs[2]
        w_hbm = refs[3] if has_w else None
        zeros_hbm = refs[ninp - 1]
        out_hbm = refs[ninp]
        sc = refs[ninp + 1:]
        src_v = sc[0:2]
        dst_v = sc[2:4]
        w_v = sc[4:6] if has_w else [None, None]
        rows = sc[6:8] if has_w else sc[4:6]
        acc = sc[-7]
        si = sc[-6:-4]
        sg = sc[-4:-2]
        ss = sc[-2:]
        c = lax.axis_index("c")
        t = lax.axis_index("s")
        pltpu.sync_copy(zeros_hbm.at[pl.ds(t * rpt, rpt)],
                        acc.at[pl.ds(t * rpt, rpt)])
        plsc.subcore_barrier()
        base = t * per_tile // 128
        off = c * nsrc

        def fire_idx(b, g):
            sl = pl.ds(base + g * kk, kk)
            pltpu.async_copy(src_hbm.at[sl], src_v[b], si[b])
            pltpu.async_copy(dst_hbm.at[sl], dst_v[b], si[b])
            if has_w:
                pltpu.async_copy(w_hbm.at[sl], w_v[b], si[b])

        def wait_idx(b, g):
            sl = pl.ds(base + g * kk, kk)
            pltpu.make_async_copy(src_hbm.at[sl], src_v[b], si[b]).wait()
            pltpu.make_async_copy(dst_hbm.at[sl], dst_v[b], si[b]).wait()
            if has_w:
                pltpu.make_async_copy(w_hbm.at[sl], w_v[b], si[b]).wait()

        def drain_scat(b):
            for j in range(kk):
                pltpu.make_async_copy(
                    rows[b].at[pl.ds(j * 128, 128)],
                    acc.at[dst_v[b].at[j]], ss[b]).wait()

        def maybe_fire_next(b_next, g_next):
            if isinstance(g_next, int):
                if g_next < ngrp:
                    fire_idx(b_next, g_next)
            else:
                @pl.when(g_next < ngrp)
                def _():
                    fire_idx(b_next, g_next)

        def process(b, g, drain_other):
            wait_idx(b, g)
            for j in range(kk):
                for q8 in range(8):
                    sl = (pl.ds(q8 * 16, 16),)
                    src_v[b].at[j][sl] = src_v[b].at[j][sl] + off
            for j in range(kk):
                pltpu.async_copy(tab_hbm.at[src_v[b].at[j]],
                                 rows[b].at[pl.ds(j * 128, 128)], sg[b])
            if drain_other:
                drain_scat(1 - b)
            maybe_fire_next(1 - b, g + 1)

            for j in range(kk):
                pltpu.make_async_copy(
                    tab_hbm.at[src_v[b].at[j]],
                    rows[b].at[pl.ds(j * 128, 128)], sg[b]).wait()
            if has_w:
                @pl.loop(0, kk)
                def _(j):
                    for e16 in range(8):
                        wv = w_v[b].at[j][pl.ds(e16 * 16, 16)]
                        for q in range(16):
                            r = rows[b].at[j * 128 + e16 * 16 + q]
                            r[...] = r[...] * _lane_bcast(wv, q)
            for j in range(kk):
                pltpu.async_copy(rows[b].at[pl.ds(j * 128, 128)],
                                 acc.at[dst_v[b].at[j]], ss[b], add=True)

        fire_idx(0, 0)
        process(0, 0, drain_other=False)
        process(1, 1, drain_other=True)

        @pl.loop(1, ngrp // 2)
        def _(i):
            process(0, 2 * i, drain_other=True)
            process(1, 2 * i + 1, drain_other=True)

        drain_scat(1)
        plsc.subcore_barrier()
        pltpu.sync_copy(acc.at[pl.ds(t * rpt, rpt)],
                        out_hbm.at[pl.ds(c * nr + t * rpt, rpt)])

    args = [table, src2d, dst2d] + ([w2d] if has_w else []) + [zeros]
    return k(*args)


def _sc_edge_map(table, streams, fn):
    erows = streams[0].shape[0]
    per_tile = erows * 128 // _NW
    ngrp = per_tile // 1024
    ntab = table.shape[0]

    @functools.partial(
        pl.kernel,
        out_type=jax.ShapeDtypeStruct((erows, 128), jnp.float32),
        mesh=_vmesh(),
        scratch_types=(
            [pltpu.VMEM((ntab,), jnp.float32)]
            + [pltpu.VMEM((8, 128), s.dtype) for s in streams]
            + [pltpu.VMEM((8, 128), jnp.float32)]
        ),
        compiler_params=_SC_PARAMS,
    )
    def k(tab_hbm, *rest):
        stream_hbms = rest[:len(streams)]
        out_hbm = rest[len(streams)]
        tab_v = rest[len(streams) + 1]
        stream_vs = rest[len(streams) + 2:-1]
        out_v = rest[-1]
        w = _wid()
        pltpu.sync_copy(tab_hbm, tab_v)
        base = w * per_tile // 128

        @pl.loop(0, ngrp)
        def _(g):
            for sh, sv in zip(stream_hbms, stream_vs):
                pltpu.sync_copy(sh.at[pl.ds(base + g * 8, 8)], sv)
            for j in range(8):
                for kk in range(8):
                    sl = pl.ds(kk * 16, 16)
                    vecs = [sv.at[j][sl] for sv in stream_vs]
                    out_v.at[j][sl] = fn(tab_v, vecs)
            pltpu.sync_copy(out_v, out_hbm.at[pl.ds(base + g * 8, 8)])

    return k(table, *streams)


def _lane_bcast(v, q):
    idx = jnp.full((16, 1), q, jnp.int32)
    dn = lax.GatherDimensionNumbers(
        offset_dims=(), collapsed_slice_dims=(0,), start_index_map=(0,))
    return lax.gather(v, idx, dn, (1,),
                      mode=lax.GatherScatterMode.PROMISE_IN_BOUNDS)


def _sc_gather_scale_scatter(table, src2d, dst2d, w2d, nr):
    nsrc = table.shape[0] // 2
    etot = src2d.shape[0] * 128
    per_tile = etot // _NS
    ngrp = per_tile // 1024
    rpt = nr // _NS
    zeros = jnp.zeros((nr, 16), jnp.float32)

    @functools.partial(
        pl.kernel,
        out_type=jax.ShapeDtypeStruct((2 * nr, 16), jnp.float32),
        mesh=_vmesh(),
        scratch_types=[
            pltpu.VMEM((8, 128), jnp.int32),
            pltpu.VMEM((8, 128), jnp.int32),
            pltpu.VMEM((8, 128), jnp.float32),
            pltpu.VMEM((128, 16), jnp.float32),
            pltpu.VMEM_SHARED((nr, 16), jnp.float32),
            pltpu.SemaphoreType.DMA,
        ],
        compiler_params=_SC_PARAMS,
    )
    def k(tab_hbm, src_hbm, dst_hbm, w_hbm, zeros_hbm, out_hbm,
          src_v, dst_v, w_v, rows_v, acc, sem):
        c = lax.axis_index("c")
        t = lax.axis_index("s")
        pltpu.sync_copy(zeros_hbm.at[pl.ds(t * rpt, rpt)],
                        acc.at[pl.ds(t * rpt, rpt)])
        plsc.subcore_barrier()
        base = t * per_tile // 128
        off = c * nsrc

        @pl.loop(0, ngrp)
        def _(g):
            pltpu.sync_copy(src_hbm.at[pl.ds(base + g * 8, 8)], src_v)
            pltpu.sync_copy(dst_hbm.at[pl.ds(base + g * 8, 8)], dst_v)
            pltpu.sync_copy(w_hbm.at[pl.ds(base + g * 8, 8)], w_v)

            @pl.loop(0, 8)
            def _(j):
                for kk in range(8):
                    sl = pl.ds(kk * 16, 16)
                    src_v.at[j][sl] = src_v.at[j][sl] + off
                pltpu.async_copy(tab_hbm.at[src_v.at[j]], rows_v, sem).wait()
                for e16 in range(8):
                    wv = w_v.at[j][pl.ds(e16 * 16, 16)]
                    for q in range(16):
                        e = e16 * 16 + q
                        rows_v.at[e][...] = (
                            rows_v.at[e][...] * _lane_bcast(wv, q))
                pltpu.sync_copy(rows_v, acc.at[dst_v.at[j]], add=True)

        plsc.subcore_barrier()
        pltpu.sync_copy(acc.at[pl.ds(t * rpt, rpt)],
                        out_hbm.at[pl.ds(c * nr + t * rpt, rpt)])

    return k(table, src2d, dst2d, w2d, zeros)


_NH_SEG = 102400
_NS_SEG = 10112


def _sc_segsum(x2stk, seg2d):
    per_tile = _NH_SEG // _NS
    rpt = _NS_SEG // _NS
    zeros = jnp.zeros((_NS_SEG, 16), jnp.float32)

    @functools.partial(
        pl.kernel,
        out_type=jax.ShapeDtypeStruct((2 * _NS_SEG, 16), jnp.float32),
        mesh=_vmesh(),
        scratch_types=[
            pltpu.VMEM((5, 128), jnp.int32),
            pltpu.VMEM((640, 16), jnp.float32),
            pltpu.VMEM_SHARED((_NS_SEG, 16), jnp.float32),
        ],
        compiler_params=_SC_PARAMS,
    )
    def k(x_hbm, seg_hbm, zeros_hbm, out_hbm, seg_v, rows_v, acc):
        c = lax.axis_index("c")
        t = lax.axis_index("s")
        pltpu.sync_copy(zeros_hbm.at[pl.ds(t * rpt, rpt)],
                        acc.at[pl.ds(t * rpt, rpt)])
        plsc.subcore_barrier()
        base_r = c * _NH_SEG + t * per_tile
        base_s = t * (per_tile // 128)

        @pl.loop(0, 10)
        def _(g):
            pltpu.sync_copy(x_hbm.at[pl.ds(base_r + g * 640, 640)], rows_v)
            pltpu.sync_copy(seg_hbm.at[pl.ds(base_s + g * 5, 5)], seg_v)
            for j in range(5):
                pltpu.sync_copy(rows_v.at[pl.ds(j * 128, 128)],
                                acc.at[seg_v.at[j]], add=True)

        plsc.subcore_barrier()
        pltpu.sync_copy(acc.at[pl.ds(t * rpt, rpt)],
                        out_hbm.at[pl.ds(c * _NS_SEG + t * rpt, rpt)])

    return k(x2stk, seg2d, zeros)


def _sc_gather_scatter(table, src2d, dst2d, nr, d):
    return _sc_msg_pass(table, src2d, dst2d, None, nr, d, gsz=256)


def _pad_idx(a, e_pad, fill):
    return jnp.concatenate(
        [a, jnp.full((e_pad - a.shape[0],), fill, jnp.int32)]).reshape(-1, 128)


_EP_FA = 1605632
_NP_FA = 100096
_EP_RGCN = 327680
_NR_RGCN = 20096


def kernel(num_prop_burst, cat_prop_burst, tweet_range_list, edge_index_burst,
           re_index, des, tweet, num_prop, cat_prop, edge_index_rgcn,
           edge_type, params):
    p = params
    act = jax.nn.leaky_relu

    num = act(num_prop_burst @ p['w_num'] + p['b_num'])
    cat = act(cat_prop_burst @ p['w_cat'] + p['b_cat'])
    x = jnp.concatenate([num, cat], axis=1)
    x = act(x @ p['w_tog'] + p['b_tog'])

    src_b = _pad_idx(edge_index_burst[0], _EP_FA, 0)
    dst_b = _pad_idx(edge_index_burst[1], _EP_FA, 100000)
    deg2 = _sc_count(dst_b, _NP_FA)
    deg = deg2[:_NP_FA, 0] + deg2[_NP_FA:, 0]
    dis = jnp.where(deg > 0, lax.rsqrt(jnp.maximum(deg, 1.0)), 0.0)
    dis_e = _sc_edge_map(
        dis, [src_b, dst_b],
        lambda tab, v: (plsc.load_gather(tab, [v[0]])
                        * plsc.load_gather(tab, [v[1]])))

    def faconv_layer(xin, x0):
        l_pad = jnp.concatenate(
            [xin @ p['w_att_l'], jnp.zeros((_NP_FA - 100000,), jnp.float32)])
        r_pad = jnp.concatenate(
            [xin @ p['w_att_r'], jnp.zeros((_NP_FA - 100000,), jnp.float32)])
        ls = _sc_edge_map(l_pad, [src_b],
                          lambda tab, v: plsc.load_gather(tab, [v[0]]))

        def wfn(tab, v):
            z = v[1] + plsc.load_gather(tab, [v[0]])
            return (1.0 - 2.0 / (jnp.exp(2.0 * z) + 1.0)) * v[2]

        w2d = _sc_edge_map(r_pad, [dst_b, ls, dis_e], wfn)
        xstk = jnp.concatenate([
            jnp.pad(xin[:, :16], ((0, _NP_FA - 100000), (0, 0))),
            jnp.pad(xin[:, 16:], ((0, _NP_FA - 100000), (0, 0)))], axis=0)
        out = _sc_gather_scale_scatter(xstk, src_b, dst_b, w2d, _NP_FA)
        o = jnp.concatenate(
            [out[:100000], out[_NP_FA:_NP_FA + 100000]], axis=1)
        return o + _EPS_FA * x0

    x1 = faconv_layer(x, x)
    x2 = faconv_layer(x1, x)
    x2 = (x2 ** 2 + 1e-08) ** 0.5

    hist = jnp.zeros((100000,), jnp.int32).at[tweet_range_list].add(1)
    seg = jnp.cumsum(hist) - 1
    seg = jnp.where((seg >= 0) & (seg < 10000), seg, 10000)
    seg2d = jnp.concatenate(
        [seg, jnp.full((_NH_SEG - 100000,), 10000, jnp.int32)]).reshape(-1, 128)
    x2stk = jnp.concatenate([
        jnp.pad(x2[:, :16], ((0, _NH_SEG - 100000), (0, 0))),
        jnp.pad(x2[:, 16:], ((0, _NH_SEG - 100000), (0, 0)))], axis=0)
    segsum = _sc_segsum(x2stk, seg2d)
    x3 = jnp.concatenate(
        [segsum[:10000], segsum[_NS_SEG:_NS_SEG + 10000]], axis=1)

    x3 = _sc_gather_rows(x3, re_index)
    x_burst = act(x3 @ p['w_map'] + p['b_map'])

    d = act(des @ p['w_des'] + p['b_des'])
    t = act(tweet @ p['w_tw'] + p['b_tw'])
    n = act(num_prop @ p['w_np'] + p['b_np'])
    c = act(cat_prop @ p['w_cp'] + p['b_cp'])
    xr = jnp.concatenate([d, t, n, c], axis=1)
    xr = act(xr @ p['w_in'] + p['b_in'])

    src_r = _pad_idx(edge_index_rgcn[0], _EP_RGCN, 0)
    gidx_r = _pad_idx(edge_index_rgcn[1] + edge_type * 10000, _EP_RGCN, 20000)
    cnt2 = _sc_count(gidx_r, _NR_RGCN)
    cnt = cnt2[:_NR_RGCN, 0] + cnt2[_NR_RGCN:, 0]
    c0 = jnp.maximum(cnt[:10000], 1.0)
    c1 = jnp.maximum(cnt[10000:20000], 1.0)

    def rgcn_layer(xin):
        xstk = jnp.concatenate([xin[:, :64], xin[:, 64:]], axis=0)
        s = _sc_gather_scatter(xstk, src_r, gidx_r, _NR_RGCN, 64)
        s_a, s_b = s[:_NR_RGCN], s[_NR_RGCN:]
        s0 = jnp.concatenate([s_a[:10000], s_b[:10000]], axis=1)
        s1 = jnp.concatenate([s_a[10000:20000], s_b[10000:20000]], axis=1)
        return (xin @ p['w_root'] + p['b_rgcn']
                + (s0 / c0[:, None]) @ p['w_rgcn'][0]
                + (s1 / c1[:, None]) @ p['w_rgcn'][1])

    xr = rgcn_layer(xr)
    xr = rgcn_layer(xr)
    x_rgcn = act(xr @ p['w_out1'] + p['b_out1'])

    xcat = jnp.concatenate([x_burst, x_rgcn], axis=1)
    xcat = act(xcat @ p['w_f0'] + p['b_f0'])
    return xcat @ p['w_f'] + p['b_f']


# speedup vs baseline: 27.8253x; 1.2972x over previous
import functools

import jax
import jax.numpy as jnp
from jax import lax
from jax.experimental import pallas as pl
from jax.experimental.pallas import tpu as pltpu
from jax.experimental.pallas import tpu_sc as plsc

_EPS_FA = 0.1

_NC = 2
_NS = 16
_NW = _NC * _NS


_SC_PARAMS = pltpu.CompilerParams(
    use_tc_tiling_on_sc=False, needs_layout_passes=False)


def _vmesh():
    return plsc.VectorSubcoreMesh(
        core_axis_name="c", subcore_axis_name="s",
        num_cores=_NC, num_subcores=_NS)


def _wid():
    return lax.axis_index("s") * _NC + lax.axis_index("c")


def _sc_gather_rows(table, idx, *, chunk=80):
    n, d = table.shape
    b = idx.shape[0]
    per_w = -(-b // (_NW * chunk)) * chunk
    bp = per_w * _NW
    nchunk = per_w // chunk
    idx_p = jnp.concatenate([idx, jnp.zeros((bp - b,), jnp.int32)])
    idx2 = idx_p.reshape(_NW * nchunk, chunk)

    @functools.partial(
        pl.kernel,
        out_type=jax.ShapeDtypeStruct((bp, d), jnp.float32),
        mesh=_vmesh(),
        scratch_types=[
            pltpu.VMEM((nchunk, chunk), jnp.int32),
            pltpu.VMEM((chunk, d), jnp.float32),
            pltpu.SemaphoreType.DMA,
        ],
        compiler_params=_SC_PARAMS,
    )
    def k(table_hbm, idx_hbm, out_hbm, idx_v, rows_v, sem):
        w = _wid()
        pltpu.sync_copy(idx_hbm.at[pl.ds(w * nchunk, nchunk)], idx_v)

        @pl.loop(0, nchunk)
        def _(j):
            pltpu.async_copy(table_hbm.at[idx_v.at[j]], rows_v, sem).wait()
            pltpu.sync_copy(
                rows_v, out_hbm.at[pl.ds(w * per_w + j * chunk, chunk)])

    return k(table, idx2)[:b]


def _sc_count(idx2d, nr):
    etot = idx2d.shape[0] * 128
    per_sc = etot // 2
    per_tile = per_sc // _NS
    ngrp = per_tile // 1024
    rpt = nr // _NS
    zeros = jnp.zeros((nr, 16), jnp.float32)

    @functools.partial(
        pl.kernel,
        out_type=jax.ShapeDtypeStruct((2 * nr, 16), jnp.float32),
        mesh=_vmesh(),
        scratch_types=[
            pltpu.VMEM((8, 128), jnp.int32),
            pltpu.VMEM((128, 16), jnp.float32),
            pltpu.VMEM_SHARED((nr, 16), jnp.float32),
            pltpu.SemaphoreType.DMA,
        ],
        compiler_params=_SC_PARAMS,
    )
    def k(idx_hbm, zeros_hbm, out_hbm, idx_v, ones_v, acc, sem):
        c = lax.axis_index("c")
        t = lax.axis_index("s")
        pltpu.sync_copy(zeros_hbm.at[pl.ds(t * rpt, rpt)],
                        acc.at[pl.ds(t * rpt, rpt)])

        @pl.loop(0, 128)
        def _(i):
            ones_v.at[i][...] = jnp.full((16,), 1.0, jnp.float32)

        plsc.subcore_barrier()
        base = (c * per_sc + t * per_tile) // 128

        @pl.loop(0, ngrp)
        def _(g):
            pltpu.sync_copy(idx_hbm.at[pl.ds(base + g * 8, 8)], idx_v)
            for j in range(8):
                pltpu.sync_copy(ones_v, acc.at[idx_v.at[j]], add=True)

        plsc.subcore_barrier()
        pltpu.sync_copy(acc.at[pl.ds(t * rpt, rpt)],
                        out_hbm.at[pl.ds(c * nr + t * rpt, rpt)])

    return k(idx2d, zeros)


def _sc_msg_pass(table, src2d, dst2d, w2d, nr, d, gsz):
    nsrc = table.shape[0] // 2
    etot = src2d.shape[0] * 128
    per_tile = etot // _NS
    kk = gsz // 128
    ngrp = per_tile // gsz
    assert ngrp % 2 == 0
    rpt = nr // _NS
    zeros = jnp.zeros((nr, d), jnp.float32)
    has_w = w2d is not None
    ninp = 5 if has_w else 4

    idx_bufs = [pltpu.VMEM((kk, 128), jnp.int32) for _ in range(4)]
    w_bufs = [pltpu.VMEM((kk, 128), jnp.float32) for _ in range(2)] \
        if has_w else []
    row_bufs = [pltpu.VMEM((gsz, d), jnp.float32) for _ in range(2)]
    sems = [pltpu.SemaphoreType.DMA for _ in range(6)]

    @functools.partial(
        pl.kernel,
        out_type=jax.ShapeDtypeStruct((2 * nr, d), jnp.float32),
        mesh=_vmesh(),
        scratch_types=(idx_bufs + w_bufs + row_bufs
                       + [pltpu.VMEM_SHARED((nr, d), jnp.float32)] + sems),
        compiler_params=_SC_PARAMS,
    )
    def k(*refs):
        tab_hbm, src_hbm, dst_hbm = refs[0], refs[1], refs[2]
        w_hbm = refs[3] if has_w else None
        zeros_hbm = refs[ninp - 1]
        out_hbm = refs[ninp]
        sc = refs[ninp + 1:]
        src_v = sc[0:2]
        dst_v = sc[2:4]
        w_v = sc[4:6] if has_w else [None, None]
        rows = sc[6:8] if has_w else sc[4:6]
        acc = sc[-7]
        si = sc[-6:-4]
        sg = sc[-4:-2]
        ss = sc[-2:]
        c = lax.axis_index("c")
        t = lax.axis_index("s")
        pltpu.sync_copy(zeros_hbm.at[pl.ds(t * rpt, rpt)],
                        acc.at[pl.ds(t * rpt, rpt)])
        plsc.subcore_barrier()
        base = t * per_tile // 128
        off = c * nsrc

        def fire_idx(b, g):
            sl = pl.ds(base + g * kk, kk)
            pltpu.async_copy(src_hbm.at[sl], src_v[b], si[b])
            pltpu.async_copy(dst_hbm.at[sl], dst_v[b], si[b])
            if has_w:
                pltpu.async_copy(w_hbm.at[sl], w_v[b], si[b])

        def wait_idx(b, g):
            sl = pl.ds(base + g * kk, kk)
            pltpu.make_async_copy(src_hbm.at[sl], src_v[b], si[b]).wait()
            pltpu.make_async_copy(dst_hbm.at[sl], dst_v[b], si[b]).wait()
            if has_w:
                pltpu.make_async_copy(w_hbm.at[sl], w_v[b], si[b]).wait()

        def drain_scat(b):
            for j in range(kk):
                pltpu.make_async_copy(
                    rows[b].at[pl.ds(j * 128, 128)],
                    acc.at[dst_v[b].at[j]], ss[b]).wait()

        def maybe_fire_next(b_next, g_next):
            if isinstance(g_next, int):
                if g_next < ngrp:
                    fire_idx(b_next, g_next)
            else:
                @pl.when(g_next < ngrp)
                def _():
                    fire_idx(b_next, g_next)

        def process(b, g, drain_other):
            wait_idx(b, g)
            for j in range(kk):
                for q8 in range(8):
                    sl = (pl.ds(q8 * 16, 16),)
                    src_v[b].at[j][sl] = src_v[b].at[j][sl] + off
            for j in range(kk):
                pltpu.async_copy(tab_hbm.at[src_v[b].at[j]],
                                 rows[b].at[pl.ds(j * 128, 128)], sg[b])
            if drain_other:
                drain_scat(1 - b)
            maybe_fire_next(1 - b, g + 1)

            for j in range(kk):
                pltpu.make_async_copy(
                    tab_hbm.at[src_v[b].at[j]],
                    rows[b].at[pl.ds(j * 128, 128)], sg[b]).wait()
            if has_w:
                @pl.loop(0, kk)
                def _(j):
                    for e16 in range(8):
                        wv = w_v[b].at[j][pl.ds(e16 * 16, 16)]
                        for q in range(16):
                            r = rows[b].at[j * 128 + e16 * 16 + q]
                            r[...] = r[...] * _lane_bcast(wv, q)
            for j in range(kk):
                pltpu.async_copy(rows[b].at[pl.ds(j * 128, 128)],
                                 acc.at[dst_v[b].at[j]], ss[b], add=True)

        fire_idx(0, 0)
        process(0, 0, drain_other=False)
        process(1, 1, drain_other=True)

        @pl.loop(1, ngrp // 2)
        def _(i):
            process(0, 2 * i, drain_other=True)
            process(1, 2 * i + 1, drain_other=True)

        drain_scat(1)
        plsc.subcore_barrier()
        pltpu.sync_copy(acc.at[pl.ds(t * rpt, rpt)],
                        out_hbm.at[pl.ds(c * nr + t * rpt, rpt)])

    args = [table, src2d, dst2d] + ([w2d] if has_w else []) + [zeros]
    return k(*args)


def _sc_edge_map(table, streams, fn):
    erows = streams[0].shape[0]
    per_tile = erows * 128 // _NW
    ngrp = per_tile // 1024
    ntab = table.shape[0]

    @functools.partial(
        pl.kernel,
        out_type=jax.ShapeDtypeStruct((erows, 128), jnp.float32),
        mesh=_vmesh(),
        scratch_types=(
            [pltpu.VMEM((ntab,), jnp.float32)]
            + [pltpu.VMEM((8, 128), s.dtype) for s in streams]
            + [pltpu.VMEM((8, 128), jnp.float32)]
        ),
        compiler_params=_SC_PARAMS,
    )
    def k(tab_hbm, *rest):
        stream_hbms = rest[:len(streams)]
        out_hbm = rest[len(streams)]
        tab_v = rest[len(streams) + 1]
        stream_vs = rest[len(streams) + 2:-1]
        out_v = rest[-1]
        w = _wid()
        pltpu.sync_copy(tab_hbm, tab_v)
        base = w * per_tile // 128

        @pl.loop(0, ngrp)
        def _(g):
            for sh, sv in zip(stream_hbms, stream_vs):
                pltpu.sync_copy(sh.at[pl.ds(base + g * 8, 8)], sv)
            for j in range(8):
                for kk in range(8):
                    sl = pl.ds(kk * 16, 16)
                    vecs = [sv.at[j][sl] for sv in stream_vs]
                    out_v.at[j][sl] = fn(tab_v, vecs)
            pltpu.sync_copy(out_v, out_hbm.at[pl.ds(base + g * 8, 8)])

    return k(table, *streams)


def _lane_bcast(v, q):
    idx = jnp.full((16, 1), q, jnp.int32)
    dn = lax.GatherDimensionNumbers(
        offset_dims=(), collapsed_slice_dims=(0,), start_index_map=(0,))
    return lax.gather(v, idx, dn, (1,),
                      mode=lax.GatherScatterMode.PROMISE_IN_BOUNDS)


def _sc_gather_scale_scatter(table, src2d, dst2d, w2d, nr):
    nsrc = table.shape[0] // 2
    etot = src2d.shape[0] * 128
    per_tile = etot // _NS
    ngrp = per_tile // 1024
    rpt = nr // _NS
    zeros = jnp.zeros((nr, 16), jnp.float32)

    @functools.partial(
        pl.kernel,
        out_type=jax.ShapeDtypeStruct((2 * nr, 16), jnp.float32),
        mesh=_vmesh(),
        scratch_types=[
            pltpu.VMEM((8, 128), jnp.int32),
            pltpu.VMEM((8, 128), jnp.int32),
            pltpu.VMEM((8, 128), jnp.float32),
            pltpu.VMEM((128, 16), jnp.float32),
            pltpu.VMEM_SHARED((nr, 16), jnp.float32),
            pltpu.SemaphoreType.DMA,
        ],
        compiler_params=_SC_PARAMS,
    )
    def k(tab_hbm, src_hbm, dst_hbm, w_hbm, zeros_hbm, out_hbm,
          src_v, dst_v, w_v, rows_v, acc, sem):
        c = lax.axis_index("c")
        t = lax.axis_index("s")
        pltpu.sync_copy(zeros_hbm.at[pl.ds(t * rpt, rpt)],
                        acc.at[pl.ds(t * rpt, rpt)])
        plsc.subcore_barrier()
        base = t * per_tile // 128
        off = c * nsrc

        @pl.loop(0, ngrp)
        def _(g):
            pltpu.sync_copy(src_hbm.at[pl.ds(base + g * 8, 8)], src_v)
            pltpu.sync_copy(dst_hbm.at[pl.ds(base + g * 8, 8)], dst_v)
            pltpu.sync_copy(w_hbm.at[pl.ds(base + g * 8, 8)], w_v)

            @pl.loop(0, 8)
            def _(j):
                for kk in range(8):
                    sl = pl.ds(kk * 16, 16)
                    src_v.at[j][sl] = src_v.at[j][sl] + off
                pltpu.async_copy(tab_hbm.at[src_v.at[j]], rows_v, sem).wait()
                for e16 in range(8):
                    wv = w_v.at[j][pl.ds(e16 * 16, 16)]
                    for q in range(16):
                        e = e16 * 16 + q
                        rows_v.at[e][...] = (
                            rows_v.at[e][...] * _lane_bcast(wv, q))
                pltpu.sync_copy(rows_v, acc.at[dst_v.at[j]], add=True)

        plsc.subcore_barrier()
        pltpu.sync_copy(acc.at[pl.ds(t * rpt, rpt)],
                        out_hbm.at[pl.ds(c * nr + t * rpt, rpt)])

    return k(table, src2d, dst2d, w2d, zeros)


_NH_SEG = 102400
_NS_SEG = 10112


def _sc_segsum(x2stk, seg2d):
    per_tile = _NH_SEG // _NS
    rpt = _NS_SEG // _NS
    zeros = jnp.zeros((_NS_SEG, 16), jnp.float32)

    @functools.partial(
        pl.kernel,
        out_type=jax.ShapeDtypeStruct((2 * _NS_SEG, 16), jnp.float32),
        mesh=_vmesh(),
        scratch_types=[
            pltpu.VMEM((5, 128), jnp.int32),
            pltpu.VMEM((640, 16), jnp.float32),
            pltpu.VMEM_SHARED((_NS_SEG, 16), jnp.float32),
        ],
        compiler_params=_SC_PARAMS,
    )
    def k(x_hbm, seg_hbm, zeros_hbm, out_hbm, seg_v, rows_v, acc):
        c = lax.axis_index("c")
        t = lax.axis_index("s")
        pltpu.sync_copy(zeros_hbm.at[pl.ds(t * rpt, rpt)],
                        acc.at[pl.ds(t * rpt, rpt)])
        plsc.subcore_barrier()
        base_r = c * _NH_SEG + t * per_tile
        base_s = t * (per_tile // 128)

        @pl.loop(0, 10)
        def _(g):
            pltpu.sync_copy(x_hbm.at[pl.ds(base_r + g * 640, 640)], rows_v)
            pltpu.sync_copy(seg_hbm.at[pl.ds(base_s + g * 5, 5)], seg_v)
            for j in range(5):
                pltpu.sync_copy(rows_v.at[pl.ds(j * 128, 128)],
                                acc.at[seg_v.at[j]], add=True)

        plsc.subcore_barrier()
        pltpu.sync_copy(acc.at[pl.ds(t * rpt, rpt)],
                        out_hbm.at[pl.ds(c * _NS_SEG + t * rpt, rpt)])

    return k(x2stk, seg2d, zeros)


def _sc_gather_scatter(table, src2d, dst2d, nr, d):
    return _sc_msg_pass(table, src2d, dst2d, None, nr, d, gsz=256)


def _pad_idx(a, e_pad, fill):
    return jnp.concatenate(
        [a, jnp.full((e_pad - a.shape[0],), fill, jnp.int32)]).reshape(-1, 128)


_EP_FA = 1605632
_NP_FA = 100096
_EP_RGCN = 327680
_NR_RGCN = 20096


def kernel(num_prop_burst, cat_prop_burst, tweet_range_list, edge_index_burst,
           re_index, des, tweet, num_prop, cat_prop, edge_index_rgcn,
           edge_type, params):
    p = params
    act = jax.nn.leaky_relu

    num = act(num_prop_burst @ p['w_num'] + p['b_num'])
    cat = act(cat_prop_burst @ p['w_cat'] + p['b_cat'])
    x = jnp.concatenate([num, cat], axis=1)
    x = act(x @ p['w_tog'] + p['b_tog'])

    src_b = _pad_idx(edge_index_burst[0], _EP_FA, 0)
    dst_b = _pad_idx(edge_index_burst[1], _EP_FA, 100000)
    deg2 = _sc_count(dst_b, _NP_FA)
    deg = deg2[:_NP_FA, 0] + deg2[_NP_FA:, 0]
    dis = jnp.where(deg > 0, lax.rsqrt(jnp.maximum(deg, 1.0)), 0.0)
    dis_e = _sc_edge_map(
        dis, [src_b, dst_b],
        lambda tab, v: (plsc.load_gather(tab, [v[0]])
                        * plsc.load_gather(tab, [v[1]])))

    def faconv_layer(xin, x0):
        l_pad = jnp.concatenate(
            [xin @ p['w_att_l'], jnp.zeros((_NP_FA - 100000,), jnp.float32)])
        r_pad = jnp.concatenate(
            [xin @ p['w_att_r'], jnp.zeros((_NP_FA - 100000,), jnp.float32)])
        ls = _sc_edge_map(l_pad, [src_b],
                          lambda tab, v: plsc.load_gather(tab, [v[0]]))

        def wfn(tab, v):
            z = v[1] + plsc.load_gather(tab, [v[0]])
            return (1.0 - 2.0 / (jnp.exp(2.0 * z) + 1.0)) * v[2]

        w2d = _sc_edge_map(r_pad, [dst_b, ls, dis_e], wfn)
        xstk = jnp.concatenate([
            jnp.pad(xin[:, :16], ((0, _NP_FA - 100000), (0, 0))),
            jnp.pad(xin[:, 16:], ((0, _NP_FA - 100000), (0, 0)))], axis=0)
        out = _sc_msg_pass(xstk, src_b, dst_b, w2d, _NP_FA, 16, gsz=256)
        o = jnp.concatenate(
            [out[:100000], out[_NP_FA:_NP_FA + 100000]], axis=1)
        return o + _EPS_FA * x0

    x1 = faconv_layer(x, x)
    x2 = faconv_layer(x1, x)
    x2 = (x2 ** 2 + 1e-08) ** 0.5

    hist = jnp.zeros((100000,), jnp.int32).at[tweet_range_list].add(1)
    seg = jnp.cumsum(hist) - 1
    seg = jnp.where((seg >= 0) & (seg < 10000), seg, 10000)
    seg2d = jnp.concatenate(
        [seg, jnp.full((_NH_SEG - 100000,), 10000, jnp.int32)]).reshape(-1, 128)
    x2stk = jnp.concatenate([
        jnp.pad(x2[:, :16], ((0, _NH_SEG - 100000), (0, 0))),
        jnp.pad(x2[:, 16:], ((0, _NH_SEG - 100000), (0, 0)))], axis=0)
    segsum = _sc_segsum(x2stk, seg2d)
    x3 = jnp.concatenate(
        [segsum[:10000], segsum[_NS_SEG:_NS_SEG + 10000]], axis=1)

    x3 = _sc_gather_rows(x3, re_index)
    x_burst = act(x3 @ p['w_map'] + p['b_map'])

    d = act(des @ p['w_des'] + p['b_des'])
    t = act(tweet @ p['w_tw'] + p['b_tw'])
    n = act(num_prop @ p['w_np'] + p['b_np'])
    c = act(cat_prop @ p['w_cp'] + p['b_cp'])
    xr = jnp.concatenate([d, t, n, c], axis=1)
    xr = act(xr @ p['w_in'] + p['b_in'])

    src_r = _pad_idx(edge_index_rgcn[0], _EP_RGCN, 0)
    gidx_r = _pad_idx(edge_index_rgcn[1] + edge_type * 10000, _EP_RGCN, 20000)
    cnt2 = _sc_count(gidx_r, _NR_RGCN)
    cnt = cnt2[:_NR_RGCN, 0] + cnt2[_NR_RGCN:, 0]
    c0 = jnp.maximum(cnt[:10000], 1.0)
    c1 = jnp.maximum(cnt[10000:20000], 1.0)

    def rgcn_layer(xin):
        xstk = jnp.concatenate([xin[:, :64], xin[:, 64:]], axis=0)
        s = _sc_gather_scatter(xstk, src_r, gidx_r, _NR_RGCN, 64)
        s_a, s_b = s[:_NR_RGCN], s[_NR_RGCN:]
        s0 = jnp.concatenate([s_a[:10000], s_b[:10000]], axis=1)
        s1 = jnp.concatenate([s_a[10000:20000], s_b[10000:20000]], axis=1)
        return (xin @ p['w_root'] + p['b_rgcn']
                + (s0 / c0[:, None]) @ p['w_rgcn'][0]
                + (s1 / c1[:, None]) @ p['w_rgcn'][1])

    xr = rgcn_layer(xr)
    xr = rgcn_layer(xr)
    x_rgcn = act(xr @ p['w_out1'] + p['b_out1'])

    xcat = jnp.concatenate([x_burst, x_rgcn], axis=1)
    xcat = act(xcat @ p['w_f0'] + p['b_f0'])
    return xcat @ p['w_f'] + p['b_f']


# speedup vs baseline: 29.8904x; 1.0742x over previous
import functools

import jax
import jax.numpy as jnp
from jax import lax
from jax.experimental import pallas as pl
from jax.experimental.pallas import tpu as pltpu
from jax.experimental.pallas import tpu_sc as plsc

_EPS_FA = 0.1

_NC = 2
_NS = 16
_NW = _NC * _NS


_SC_PARAMS = pltpu.CompilerParams(
    use_tc_tiling_on_sc=False, needs_layout_passes=False)


def _vmesh():
    return plsc.VectorSubcoreMesh(
        core_axis_name="c", subcore_axis_name="s",
        num_cores=_NC, num_subcores=_NS)


def _wid():
    return lax.axis_index("s") * _NC + lax.axis_index("c")


def _sc_gather_rows(table, idx, *, chunk=80):
    n, d = table.shape
    b = idx.shape[0]
    per_w = -(-b // (_NW * chunk)) * chunk
    bp = per_w * _NW
    nchunk = per_w // chunk
    idx_p = jnp.concatenate([idx, jnp.zeros((bp - b,), jnp.int32)])
    idx2 = idx_p.reshape(_NW * nchunk, chunk)

    @functools.partial(
        pl.kernel,
        out_type=jax.ShapeDtypeStruct((bp, d), jnp.float32),
        mesh=_vmesh(),
        scratch_types=[
            pltpu.VMEM((nchunk, chunk), jnp.int32),
            pltpu.VMEM((chunk, d), jnp.float32),
            pltpu.SemaphoreType.DMA,
        ],
        compiler_params=_SC_PARAMS,
    )
    def k(table_hbm, idx_hbm, out_hbm, idx_v, rows_v, sem):
        w = _wid()
        pltpu.sync_copy(idx_hbm.at[pl.ds(w * nchunk, nchunk)], idx_v)

        @pl.loop(0, nchunk)
        def _(j):
            pltpu.async_copy(table_hbm.at[idx_v.at[j]], rows_v, sem).wait()
            pltpu.sync_copy(
                rows_v, out_hbm.at[pl.ds(w * per_w + j * chunk, chunk)])

    return k(table, idx2)[:b]


def _sc_count(idx2d, nr):
    etot = idx2d.shape[0] * 128
    per_sc = etot // 2
    per_tile = per_sc // _NS
    ngrp = per_tile // 1024
    rpt = nr // _NS
    zeros = jnp.zeros((nr, 16), jnp.float32)

    @functools.partial(
        pl.kernel,
        out_type=jax.ShapeDtypeStruct((2 * nr, 16), jnp.float32),
        mesh=_vmesh(),
        scratch_types=[
            pltpu.VMEM((8, 128), jnp.int32),
            pltpu.VMEM((128, 16), jnp.float32),
            pltpu.VMEM_SHARED((nr, 16), jnp.float32),
            pltpu.SemaphoreType.DMA,
        ],
        compiler_params=_SC_PARAMS,
    )
    def k(idx_hbm, zeros_hbm, out_hbm, idx_v, ones_v, acc, sem):
        c = lax.axis_index("c")
        t = lax.axis_index("s")
        pltpu.sync_copy(zeros_hbm.at[pl.ds(t * rpt, rpt)],
                        acc.at[pl.ds(t * rpt, rpt)])

        @pl.loop(0, 128)
        def _(i):
            ones_v.at[i][...] = jnp.full((16,), 1.0, jnp.float32)

        plsc.subcore_barrier()
        base = (c * per_sc + t * per_tile) // 128

        @pl.loop(0, ngrp)
        def _(g):
            pltpu.sync_copy(idx_hbm.at[pl.ds(base + g * 8, 8)], idx_v)
            for j in range(8):
                pltpu.sync_copy(ones_v, acc.at[idx_v.at[j]], add=True)

        plsc.subcore_barrier()
        pltpu.sync_copy(acc.at[pl.ds(t * rpt, rpt)],
                        out_hbm.at[pl.ds(c * nr + t * rpt, rpt)])

    return k(idx2d, zeros)


def _sc_msg_pass(table, src2d, dst2d, w2d, nr, d, gsz):
    nsrc = table.shape[0] // 2
    etot = src2d.shape[0] * 128
    per_tile = etot // _NS
    kk = gsz // 128
    ngrp = per_tile // gsz
    assert ngrp % 2 == 0
    rpt = nr // _NS
    zeros = jnp.zeros((nr, d), jnp.float32)
    has_w = w2d is not None
    ninp = 5 if has_w else 4

    idx_bufs = [pltpu.VMEM((kk, 128), jnp.int32) for _ in range(4)]
    w_bufs = [pltpu.VMEM((kk, 128), jnp.float32) for _ in range(2)] \
        if has_w else []
    row_bufs = [pltpu.VMEM((gsz, d), jnp.float32) for _ in range(2)]
    sems = [pltpu.SemaphoreType.DMA for _ in range(6)]

    @functools.partial(
        pl.kernel,
        out_type=jax.ShapeDtypeStruct((2 * nr, d), jnp.float32),
        mesh=_vmesh(),
        scratch_types=(idx_bufs + w_bufs + row_bufs
                       + [pltpu.VMEM_SHARED((nr, d), jnp.float32)] + sems),
        compiler_params=_SC_PARAMS,
    )
    def k(*refs):
        tab_hbm, src_hbm, dst_hbm = refs[0], refs[1], refs[2]
        w_hbm = refs[3] if has_w else None
        zeros_hbm = refs[ninp - 1]
        out_hbm = refs[ninp]
        sc = refs[ninp + 1:]
        src_v = sc[0:2]
        dst_v = sc[2:4]
        w_v = sc[4:6] if has_w else [None, None]
        rows = sc[6:8] if has_w else sc[4:6]
        acc = sc[-7]
        si = sc[-6:-4]
        sg = sc[-4:-2]
        ss = sc[-2:]
        c = lax.axis_index("c")
        t = lax.axis_index("s")
        pltpu.sync_copy(zeros_hbm.at[pl.ds(t * rpt, rpt)],
                        acc.at[pl.ds(t * rpt, rpt)])
        plsc.subcore_barrier()
        base = t * per_tile // 128
        off = c * nsrc

        def fire_idx(b, g):
            sl = pl.ds(base + g * kk, kk)
            pltpu.async_copy(src_hbm.at[sl], src_v[b], si[b])
            pltpu.async_copy(dst_hbm.at[sl], dst_v[b], si[b])
            if has_w:
                pltpu.async_copy(w_hbm.at[sl], w_v[b], si[b])

        def wait_idx(b, g):
            sl = pl.ds(base + g * kk, kk)
            pltpu.make_async_copy(src_hbm.at[sl], src_v[b], si[b]).wait()
            pltpu.make_async_copy(dst_hbm.at[sl], dst_v[b], si[b]).wait()
            if has_w:
                pltpu.make_async_copy(w_hbm.at[sl], w_v[b], si[b]).wait()

        def drain_scat(b):
            for j in range(kk):
                pltpu.make_async_copy(
                    rows[b].at[pl.ds(j * 128, 128)],
                    acc.at[dst_v[b].at[j]], ss[b]).wait()

        def maybe_fire_next(b_next, g_next):
            if isinstance(g_next, int):
                if g_next < ngrp:
                    fire_idx(b_next, g_next)
            else:
                @pl.when(g_next < ngrp)
                def _():
                    fire_idx(b_next, g_next)

        def process(b, g, drain_other):
            wait_idx(b, g)
            for j in range(kk):
                for q8 in range(8):
                    sl = (pl.ds(q8 * 16, 16),)
                    src_v[b].at[j][sl] = src_v[b].at[j][sl] + off
            for j in range(kk):
                pltpu.async_copy(tab_hbm.at[src_v[b].at[j]],
                                 rows[b].at[pl.ds(j * 128, 128)], sg[b])
            if drain_other:
                drain_scat(1 - b)
            maybe_fire_next(1 - b, g + 1)

            for j in range(kk):
                pltpu.make_async_copy(
                    tab_hbm.at[src_v[b].at[j]],
                    rows[b].at[pl.ds(j * 128, 128)], sg[b]).wait()
            if has_w:
                @pl.loop(0, kk)
                def _(j):
                    for e16 in range(8):
                        wv = w_v[b].at[j][pl.ds(e16 * 16, 16)]
                        for q in range(16):
                            r = rows[b].at[j * 128 + e16 * 16 + q]
                            r[...] = r[...] * _lane_bcast(wv, q)
            for j in range(kk):
                pltpu.async_copy(rows[b].at[pl.ds(j * 128, 128)],
                                 acc.at[dst_v[b].at[j]], ss[b], add=True)

        fire_idx(0, 0)
        process(0, 0, drain_other=False)
        process(1, 1, drain_other=True)

        @pl.loop(1, ngrp // 2)
        def _(i):
            process(0, 2 * i, drain_other=True)
            process(1, 2 * i + 1, drain_other=True)

        drain_scat(1)
        plsc.subcore_barrier()
        pltpu.sync_copy(acc.at[pl.ds(t * rpt, rpt)],
                        out_hbm.at[pl.ds(c * nr + t * rpt, rpt)])

    args = [table, src2d, dst2d] + ([w2d] if has_w else []) + [zeros]
    return k(*args)


def _sc_edge_map(table, streams, fn):
    erows = streams[0].shape[0]
    per_tile = erows * 128 // _NW
    ngrp = per_tile // 1024
    ntab = table.shape[0]

    @functools.partial(
        pl.kernel,
        out_type=jax.ShapeDtypeStruct((erows, 128), jnp.float32),
        mesh=_vmesh(),
        scratch_types=(
            [pltpu.VMEM((ntab,), jnp.float32)]
            + [pltpu.VMEM((8, 128), s.dtype) for s in streams]
            + [pltpu.VMEM((8, 128), jnp.float32)]
        ),
        compiler_params=_SC_PARAMS,
    )
    def k(tab_hbm, *rest):
        stream_hbms = rest[:len(streams)]
        out_hbm = rest[len(streams)]
        tab_v = rest[len(streams) + 1]
        stream_vs = rest[len(streams) + 2:-1]
        out_v = rest[-1]
        w = _wid()
        pltpu.sync_copy(tab_hbm, tab_v)
        base = w * per_tile // 128

        @pl.loop(0, ngrp)
        def _(g):
            for sh, sv in zip(stream_hbms, stream_vs):
                pltpu.sync_copy(sh.at[pl.ds(base + g * 8, 8)], sv)
            for j in range(8):
                for kk in range(8):
                    sl = pl.ds(kk * 16, 16)
                    vecs = [sv.at[j][sl] for sv in stream_vs]
                    out_v.at[j][sl] = fn(tab_v, vecs)
            pltpu.sync_copy(out_v, out_hbm.at[pl.ds(base + g * 8, 8)])

    return k(table, *streams)


def _lane_bcast(v, q):
    idx = jnp.full((16, 1), q, jnp.int32)
    dn = lax.GatherDimensionNumbers(
        offset_dims=(), collapsed_slice_dims=(0,), start_index_map=(0,))
    return lax.gather(v, idx, dn, (1,),
                      mode=lax.GatherScatterMode.PROMISE_IN_BOUNDS)


def _sc_gather_scale_scatter(table, src2d, dst2d, w2d, nr):
    nsrc = table.shape[0] // 2
    etot = src2d.shape[0] * 128
    per_tile = etot // _NS
    ngrp = per_tile // 1024
    rpt = nr // _NS
    zeros = jnp.zeros((nr, 16), jnp.float32)

    @functools.partial(
        pl.kernel,
        out_type=jax.ShapeDtypeStruct((2 * nr, 16), jnp.float32),
        mesh=_vmesh(),
        scratch_types=[
            pltpu.VMEM((8, 128), jnp.int32),
            pltpu.VMEM((8, 128), jnp.int32),
            pltpu.VMEM((8, 128), jnp.float32),
            pltpu.VMEM((128, 16), jnp.float32),
            pltpu.VMEM_SHARED((nr, 16), jnp.float32),
            pltpu.SemaphoreType.DMA,
        ],
        compiler_params=_SC_PARAMS,
    )
    def k(tab_hbm, src_hbm, dst_hbm, w_hbm, zeros_hbm, out_hbm,
          src_v, dst_v, w_v, rows_v, acc, sem):
        c = lax.axis_index("c")
        t = lax.axis_index("s")
        pltpu.sync_copy(zeros_hbm.at[pl.ds(t * rpt, rpt)],
                        acc.at[pl.ds(t * rpt, rpt)])
        plsc.subcore_barrier()
        base = t * per_tile // 128
        off = c * nsrc

        @pl.loop(0, ngrp)
        def _(g):
            pltpu.sync_copy(src_hbm.at[pl.ds(base + g * 8, 8)], src_v)
            pltpu.sync_copy(dst_hbm.at[pl.ds(base + g * 8, 8)], dst_v)
            pltpu.sync_copy(w_hbm.at[pl.ds(base + g * 8, 8)], w_v)

            @pl.loop(0, 8)
            def _(j):
                for kk in range(8):
                    sl = pl.ds(kk * 16, 16)
                    src_v.at[j][sl] = src_v.at[j][sl] + off
                pltpu.async_copy(tab_hbm.at[src_v.at[j]], rows_v, sem).wait()
                for e16 in range(8):
                    wv = w_v.at[j][pl.ds(e16 * 16, 16)]
                    for q in range(16):
                        e = e16 * 16 + q
                        rows_v.at[e][...] = (
                            rows_v.at[e][...] * _lane_bcast(wv, q))
                pltpu.sync_copy(rows_v, acc.at[dst_v.at[j]], add=True)

        plsc.subcore_barrier()
        pltpu.sync_copy(acc.at[pl.ds(t * rpt, rpt)],
                        out_hbm.at[pl.ds(c * nr + t * rpt, rpt)])

    return k(table, src2d, dst2d, w2d, zeros)


_NH_SEG = 102400
_NS_SEG = 10112


def _sc_segsum(x2stk, seg2d):
    per_tile = _NH_SEG // _NS
    rpt = _NS_SEG // _NS
    zeros = jnp.zeros((_NS_SEG, 16), jnp.float32)

    @functools.partial(
        pl.kernel,
        out_type=jax.ShapeDtypeStruct((2 * _NS_SEG, 16), jnp.float32),
        mesh=_vmesh(),
        scratch_types=[
            pltpu.VMEM((5, 128), jnp.int32),
            pltpu.VMEM((640, 16), jnp.float32),
            pltpu.VMEM_SHARED((_NS_SEG, 16), jnp.float32),
        ],
        compiler_params=_SC_PARAMS,
    )
    def k(x_hbm, seg_hbm, zeros_hbm, out_hbm, seg_v, rows_v, acc):
        c = lax.axis_index("c")
        t = lax.axis_index("s")
        pltpu.sync_copy(zeros_hbm.at[pl.ds(t * rpt, rpt)],
                        acc.at[pl.ds(t * rpt, rpt)])
        plsc.subcore_barrier()
        base_r = c * _NH_SEG + t * per_tile
        base_s = t * (per_tile // 128)

        @pl.loop(0, 10)
        def _(g):
            pltpu.sync_copy(x_hbm.at[pl.ds(base_r + g * 640, 640)], rows_v)
            pltpu.sync_copy(seg_hbm.at[pl.ds(base_s + g * 5, 5)], seg_v)
            for j in range(5):
                pltpu.sync_copy(rows_v.at[pl.ds(j * 128, 128)],
                                acc.at[seg_v.at[j]], add=True)

        plsc.subcore_barrier()
        pltpu.sync_copy(acc.at[pl.ds(t * rpt, rpt)],
                        out_hbm.at[pl.ds(c * _NS_SEG + t * rpt, rpt)])

    return k(x2stk, seg2d, zeros)


def _sc_gather_scatter(table, src2d, dst2d, nr, d):
    return _sc_msg_pass(table, src2d, dst2d, None, nr, d, gsz=256)


def _pad_idx(a, e_pad, fill):
    return jnp.concatenate(
        [a, jnp.full((e_pad - a.shape[0],), fill, jnp.int32)]).reshape(-1, 128)


_EP_FA = 1605632
_NP_FA = 100096
_EP_RGCN = 327680
_NR_RGCN = 20096


def kernel(num_prop_burst, cat_prop_burst, tweet_range_list, edge_index_burst,
           re_index, des, tweet, num_prop, cat_prop, edge_index_rgcn,
           edge_type, params):
    p = params
    act = jax.nn.leaky_relu

    num = act(num_prop_burst @ p['w_num'] + p['b_num'])
    cat = act(cat_prop_burst @ p['w_cat'] + p['b_cat'])
    x = jnp.concatenate([num, cat], axis=1)
    x = act(x @ p['w_tog'] + p['b_tog'])

    src_b = _pad_idx(edge_index_burst[0], _EP_FA, 0)
    dst_b = _pad_idx(edge_index_burst[1], _EP_FA, 100000)
    deg2 = _sc_count(dst_b, _NP_FA)
    deg = deg2[:_NP_FA, 0] + deg2[_NP_FA:, 0]
    dis = jnp.where(deg > 0, lax.rsqrt(jnp.maximum(deg, 1.0)), 0.0)
    dis_e = _sc_edge_map(
        dis, [src_b, dst_b],
        lambda tab, v: (plsc.load_gather(tab, [v[0]])
                        * plsc.load_gather(tab, [v[1]])))

    def faconv_layer(xin, x0):
        l_pad = jnp.concatenate(
            [xin @ p['w_att_l'], jnp.zeros((_NP_FA - 100000,), jnp.float32)])
        r_pad = jnp.concatenate(
            [xin @ p['w_att_r'], jnp.zeros((_NP_FA - 100000,), jnp.float32)])
        ls = _sc_edge_map(l_pad, [src_b],
                          lambda tab, v: plsc.load_gather(tab, [v[0]]))

        def wfn(tab, v):
            z = v[1] + plsc.load_gather(tab, [v[0]])
            return (1.0 - 2.0 / (jnp.exp(2.0 * z) + 1.0)) * v[2]

        w2d = _sc_edge_map(r_pad, [dst_b, ls, dis_e], wfn)
        xstk = jnp.concatenate([
            jnp.pad(xin[:, :16], ((0, _NP_FA - 100000), (0, 0))),
            jnp.pad(xin[:, 16:], ((0, _NP_FA - 100000), (0, 0)))], axis=0)
        out = _sc_msg_pass(xstk, src_b, dst_b, w2d, _NP_FA, 16, gsz=512)
        o = jnp.concatenate(
            [out[:100000], out[_NP_FA:_NP_FA + 100000]], axis=1)
        return o + _EPS_FA * x0

    x1 = faconv_layer(x, x)
    x2 = faconv_layer(x1, x)
    x2 = (x2 ** 2 + 1e-08) ** 0.5

    hist = jnp.zeros((100000,), jnp.int32).at[tweet_range_list].add(1)
    seg = jnp.cumsum(hist) - 1
    seg = jnp.where((seg >= 0) & (seg < 10000), seg, 10000)
    seg2d = jnp.concatenate(
        [seg, jnp.full((_NH_SEG - 100000,), 10000, jnp.int32)]).reshape(-1, 128)
    x2stk = jnp.concatenate([
        jnp.pad(x2[:, :16], ((0, _NH_SEG - 100000), (0, 0))),
        jnp.pad(x2[:, 16:], ((0, _NH_SEG - 100000), (0, 0)))], axis=0)
    segsum = _sc_segsum(x2stk, seg2d)
    x3 = jnp.concatenate(
        [segsum[:10000], segsum[_NS_SEG:_NS_SEG + 10000]], axis=1)

    x3 = _sc_gather_rows(x3, re_index)
    x_burst = act(x3 @ p['w_map'] + p['b_map'])

    d = act(des @ p['w_des'] + p['b_des'])
    t = act(tweet @ p['w_tw'] + p['b_tw'])
    n = act(num_prop @ p['w_np'] + p['b_np'])
    c = act(cat_prop @ p['w_cp'] + p['b_cp'])
    xr = jnp.concatenate([d, t, n, c], axis=1)
    xr = act(xr @ p['w_in'] + p['b_in'])

    src_r = _pad_idx(edge_index_rgcn[0], _EP_RGCN, 0)
    gidx_r = _pad_idx(edge_index_rgcn[1] + edge_type * 10000, _EP_RGCN, 20000)
    cnt2 = _sc_count(gidx_r, _NR_RGCN)
    cnt = cnt2[:_NR_RGCN, 0] + cnt2[_NR_RGCN:, 0]
    c0 = jnp.maximum(cnt[:10000], 1.0)
    c1 = jnp.maximum(cnt[10000:20000], 1.0)

    def rgcn_layer(xin):
        xstk = jnp.concatenate([xin[:, :64], xin[:, 64:]], axis=0)
        s = _sc_gather_scatter(xstk, src_r, gidx_r, _NR_RGCN, 64)
        s_a, s_b = s[:_NR_RGCN], s[_NR_RGCN:]
        s0 = jnp.concatenate([s_a[:10000], s_b[:10000]], axis=1)
        s1 = jnp.concatenate([s_a[10000:20000], s_b[10000:20000]], axis=1)
        return (xin @ p['w_root'] + p['b_rgcn']
                + (s0 / c0[:, None]) @ p['w_rgcn'][0]
                + (s1 / c1[:, None]) @ p['w_rgcn'][1])

    xr = rgcn_layer(xr)
    xr = rgcn_layer(xr)
    x_rgcn = act(xr @ p['w_out1'] + p['b_out1'])

    xcat = jnp.concatenate([x_burst, x_rgcn], axis=1)
    xcat = act(xcat @ p['w_f0'] + p['b_f0'])
    return xcat @ p['w_f'] + p['b_f']


# speedup vs baseline: 31.9790x; 1.0699x over previous
import functools

import jax
import jax.numpy as jnp
from jax import lax
from jax.experimental import pallas as pl
from jax.experimental.pallas import tpu as pltpu
from jax.experimental.pallas import tpu_sc as plsc

_EPS_FA = 0.1

_NC = 2
_NS = 16
_NW = _NC * _NS


_SC_PARAMS = pltpu.CompilerParams(
    use_tc_tiling_on_sc=False, needs_layout_passes=False)


def _vmesh():
    return plsc.VectorSubcoreMesh(
        core_axis_name="c", subcore_axis_name="s",
        num_cores=_NC, num_subcores=_NS)


def _wid():
    return lax.axis_index("s") * _NC + lax.axis_index("c")


def _sc_gather_rows(table, idx, *, chunk=80):
    n, d = table.shape
    b = idx.shape[0]
    per_w = -(-b // (_NW * chunk)) * chunk
    bp = per_w * _NW
    nchunk = per_w // chunk
    idx_p = jnp.concatenate([idx, jnp.zeros((bp - b,), jnp.int32)])
    idx2 = idx_p.reshape(_NW * nchunk, chunk)

    @functools.partial(
        pl.kernel,
        out_type=jax.ShapeDtypeStruct((bp, d), jnp.float32),
        mesh=_vmesh(),
        scratch_types=[
            pltpu.VMEM((nchunk, chunk), jnp.int32),
            pltpu.VMEM((chunk, d), jnp.float32),
            pltpu.SemaphoreType.DMA,
        ],
        compiler_params=_SC_PARAMS,
    )
    def k(table_hbm, idx_hbm, out_hbm, idx_v, rows_v, sem):
        w = _wid()
        pltpu.sync_copy(idx_hbm.at[pl.ds(w * nchunk, nchunk)], idx_v)

        @pl.loop(0, nchunk)
        def _(j):
            pltpu.async_copy(table_hbm.at[idx_v.at[j]], rows_v, sem).wait()
            pltpu.sync_copy(
                rows_v, out_hbm.at[pl.ds(w * per_w + j * chunk, chunk)])

    return k(table, idx2)[:b]


def _sc_count(idx2d, nr):
    etot = idx2d.shape[0] * 128
    per_sc = etot // 2
    per_tile = per_sc // _NS
    ngrp = per_tile // 1024
    rpt = nr // _NS
    zeros = jnp.zeros((nr, 16), jnp.float32)

    @functools.partial(
        pl.kernel,
        out_type=jax.ShapeDtypeStruct((2 * nr, 16), jnp.float32),
        mesh=_vmesh(),
        scratch_types=[
            pltpu.VMEM((8, 128), jnp.int32),
            pltpu.VMEM((128, 16), jnp.float32),
            pltpu.VMEM_SHARED((nr, 16), jnp.float32),
            pltpu.SemaphoreType.DMA,
        ],
        compiler_params=_SC_PARAMS,
    )
    def k(idx_hbm, zeros_hbm, out_hbm, idx_v, ones_v, acc, sem):
        c = lax.axis_index("c")
        t = lax.axis_index("s")
        pltpu.sync_copy(zeros_hbm.at[pl.ds(t * rpt, rpt)],
                        acc.at[pl.ds(t * rpt, rpt)])

        @pl.loop(0, 128)
        def _(i):
            ones_v.at[i][...] = jnp.full((16,), 1.0, jnp.float32)

        plsc.subcore_barrier()
        base = (c * per_sc + t * per_tile) // 128

        @pl.loop(0, ngrp)
        def _(g):
            pltpu.sync_copy(idx_hbm.at[pl.ds(base + g * 8, 8)], idx_v)
            for j in range(8):
                pltpu.sync_copy(ones_v, acc.at[idx_v.at[j]], add=True)

        plsc.subcore_barrier()
        pltpu.sync_copy(acc.at[pl.ds(t * rpt, rpt)],
                        out_hbm.at[pl.ds(c * nr + t * rpt, rpt)])

    return k(idx2d, zeros)


def _sc_msg_pass(table, src2d, dst2d, w2d, nr, d, gsz):
    nsrc = table.shape[0] // 2
    etot = src2d.shape[0] * 128
    per_tile = etot // _NS
    kk = gsz // 128
    ngrp = per_tile // gsz
    assert ngrp % 2 == 0
    rpt = nr // _NS
    zeros = jnp.zeros((nr, d), jnp.float32)
    has_w = w2d is not None
    ninp = 5 if has_w else 4

    idx_bufs = [pltpu.VMEM((kk, 128), jnp.int32) for _ in range(4)]
    w_bufs = [pltpu.VMEM((kk, 128), jnp.float32) for _ in range(2)] \
        if has_w else []
    row_bufs = [pltpu.VMEM((gsz, d), jnp.float32) for _ in range(2)]
    sems = [pltpu.SemaphoreType.DMA for _ in range(6)]

    @functools.partial(
        pl.kernel,
        out_type=jax.ShapeDtypeStruct((2 * nr, d), jnp.float32),
        mesh=_vmesh(),
        scratch_types=(idx_bufs + w_bufs + row_bufs
                       + [pltpu.VMEM_SHARED((nr, d), jnp.float32)] + sems),
        compiler_params=_SC_PARAMS,
    )
    def k(*refs):
        tab_hbm, src_hbm, dst_hbm = refs[0], refs[1], refs[2]
        w_hbm = refs[3] if has_w else None
        zeros_hbm = refs[ninp - 1]
        out_hbm = refs[ninp]
        sc = refs[ninp + 1:]
        src_v = sc[0:2]
        dst_v = sc[2:4]
        w_v = sc[4:6] if has_w else [None, None]
        rows = sc[6:8] if has_w else sc[4:6]
        acc = sc[-7]
        si = sc[-6:-4]
        sg = sc[-4:-2]
        ss = sc[-2:]
        c = lax.axis_index("c")
        t = lax.axis_index("s")
        pltpu.sync_copy(zeros_hbm.at[pl.ds(t * rpt, rpt)],
                        acc.at[pl.ds(t * rpt, rpt)])
        plsc.subcore_barrier()
        base = t * per_tile // 128
        off = c * nsrc

        def fire_idx(b, g):
            sl = pl.ds(base + g * kk, kk)
            pltpu.async_copy(src_hbm.at[sl], src_v[b], si[b])
            pltpu.async_copy(dst_hbm.at[sl], dst_v[b], si[b])
            if has_w:
                pltpu.async_copy(w_hbm.at[sl], w_v[b], si[b])

        def wait_idx(b, g):
            sl = pl.ds(base + g * kk, kk)
            pltpu.make_async_copy(src_hbm.at[sl], src_v[b], si[b]).wait()
            pltpu.make_async_copy(dst_hbm.at[sl], dst_v[b], si[b]).wait()
            if has_w:
                pltpu.make_async_copy(w_hbm.at[sl], w_v[b], si[b]).wait()

        def drain_scat(b):
            for j in range(kk):
                pltpu.make_async_copy(
                    rows[b].at[pl.ds(j * 128, 128)],
                    acc.at[dst_v[b].at[j]], ss[b]).wait()

        def maybe_fire_next(b_next, g_next):
            if isinstance(g_next, int):
                if g_next < ngrp:
                    fire_idx(b_next, g_next)
            else:
                @pl.when(g_next < ngrp)
                def _():
                    fire_idx(b_next, g_next)

        def process(b, g, drain_other):
            wait_idx(b, g)
            for j in range(kk):
                for q8 in range(8):
                    sl = (pl.ds(q8 * 16, 16),)
                    src_v[b].at[j][sl] = src_v[b].at[j][sl] + off
            for j in range(kk):
                pltpu.async_copy(tab_hbm.at[src_v[b].at[j]],
                                 rows[b].at[pl.ds(j * 128, 128)], sg[b])
            if drain_other:
                drain_scat(1 - b)
            maybe_fire_next(1 - b, g + 1)

            for j in range(kk):
                pltpu.make_async_copy(
                    tab_hbm.at[src_v[b].at[j]],
                    rows[b].at[pl.ds(j * 128, 128)], sg[b]).wait()
            if has_w:
                @pl.loop(0, kk)
                def _(j):
                    for e16 in range(8):
                        wv = w_v[b].at[j][pl.ds(e16 * 16, 16)]
                        for q in range(16):
                            r = rows[b].at[j * 128 + e16 * 16 + q]
                            r[...] = r[...] * _lane_bcast(wv, q)
            for j in range(kk):
                pltpu.async_copy(rows[b].at[pl.ds(j * 128, 128)],
                                 acc.at[dst_v[b].at[j]], ss[b], add=True)

        fire_idx(0, 0)
        process(0, 0, drain_other=False)
        process(1, 1, drain_other=True)

        @pl.loop(1, ngrp // 2)
        def _(i):
            process(0, 2 * i, drain_other=True)
            process(1, 2 * i + 1, drain_other=True)

        drain_scat(1)
        plsc.subcore_barrier()
        pltpu.sync_copy(acc.at[pl.ds(t * rpt, rpt)],
                        out_hbm.at[pl.ds(c * nr + t * rpt, rpt)])

    args = [table, src2d, dst2d] + ([w2d] if has_w else []) + [zeros]
    return k(*args)


def _sc_edge_map(table, streams, fn):
    erows = streams[0].shape[0]
    per_tile = erows * 128 // _NW
    ngrp = per_tile // 1024
    ntab = table.shape[0]
    nst = len(streams)
    assert ngrp >= 2

    @functools.partial(
        pl.kernel,
        out_type=jax.ShapeDtypeStruct((erows, 128), jnp.float32),
        mesh=_vmesh(),
        scratch_types=(
            [pltpu.VMEM((ntab,), jnp.float32)]
            + [pltpu.VMEM((8, 128), s.dtype) for _ in (0, 1) for s in streams]
            + [pltpu.VMEM((8, 128), jnp.float32) for _ in (0, 1)]
            + [pltpu.SemaphoreType.DMA for _ in range(4)]
        ),
        compiler_params=_SC_PARAMS,
    )
    def k(tab_hbm, *rest):
        stream_hbms = rest[:nst]
        out_hbm = rest[nst]
        sc = rest[nst + 1:]
        tab_v = sc[0]
        sbufs = [sc[1:1 + nst], sc[1 + nst:1 + 2 * nst]]
        obufs = sc[1 + 2 * nst:3 + 2 * nst]
        si = sc[3 + 2 * nst:5 + 2 * nst]
        so = sc[5 + 2 * nst:7 + 2 * nst]
        w = _wid()
        pltpu.sync_copy(tab_hbm, tab_v)
        base = w * per_tile // 128

        def fire_in(b, g):
            for sh, sv in zip(stream_hbms, sbufs[b]):
                pltpu.async_copy(sh.at[pl.ds(base + g * 8, 8)], sv, si[b])

        def wait_in(b, g):
            for sh, sv in zip(stream_hbms, sbufs[b]):
                pltpu.make_async_copy(
                    sh.at[pl.ds(base + g * 8, 8)], sv, si[b]).wait()

        def out_sl(g):
            return out_hbm.at[pl.ds(base + g * 8, 8)]

        def process(b, g, wait_out):
            wait_in(b, g)
            if isinstance(g, int):
                if g + 1 < ngrp:
                    fire_in(1 - b, g + 1)
            else:
                @pl.when(g + 1 < ngrp)
                def _():
                    fire_in(1 - b, g + 1)
            if wait_out:
                pltpu.make_async_copy(obufs[b], out_sl(g), so[b]).wait()
            for j in range(8):
                for q in range(8):
                    sl = pl.ds(q * 16, 16)
                    vecs = [sv.at[j][sl] for sv in sbufs[b]]
                    obufs[b].at[j][sl] = fn(tab_v, vecs)
            pltpu.async_copy(obufs[b], out_sl(g), so[b])

        fire_in(0, 0)
        process(0, 0, wait_out=False)
        process(1, 1, wait_out=False)

        @pl.loop(1, ngrp // 2)
        def _(i):
            process(0, 2 * i, wait_out=True)
            process(1, 2 * i + 1, wait_out=True)

        if ngrp % 2:
            process(0, ngrp - 1, wait_out=True)
        pltpu.make_async_copy(obufs[0], out_sl(0), so[0]).wait()
        pltpu.make_async_copy(obufs[1], out_sl(1), so[1]).wait()

    return k(table, *streams)


def _lane_bcast(v, q):
    idx = jnp.full((16, 1), q, jnp.int32)
    dn = lax.GatherDimensionNumbers(
        offset_dims=(), collapsed_slice_dims=(0,), start_index_map=(0,))
    return lax.gather(v, idx, dn, (1,),
                      mode=lax.GatherScatterMode.PROMISE_IN_BOUNDS)


def _sc_gather_scale_scatter(table, src2d, dst2d, w2d, nr):
    nsrc = table.shape[0] // 2
    etot = src2d.shape[0] * 128
    per_tile = etot // _NS
    ngrp = per_tile // 1024
    rpt = nr // _NS
    zeros = jnp.zeros((nr, 16), jnp.float32)

    @functools.partial(
        pl.kernel,
        out_type=jax.ShapeDtypeStruct((2 * nr, 16), jnp.float32),
        mesh=_vmesh(),
        scratch_types=[
            pltpu.VMEM((8, 128), jnp.int32),
            pltpu.VMEM((8, 128), jnp.int32),
            pltpu.VMEM((8, 128), jnp.float32),
            pltpu.VMEM((128, 16), jnp.float32),
            pltpu.VMEM_SHARED((nr, 16), jnp.float32),
            pltpu.SemaphoreType.DMA,
        ],
        compiler_params=_SC_PARAMS,
    )
    def k(tab_hbm, src_hbm, dst_hbm, w_hbm, zeros_hbm, out_hbm,
          src_v, dst_v, w_v, rows_v, acc, sem):
        c = lax.axis_index("c")
        t = lax.axis_index("s")
        pltpu.sync_copy(zeros_hbm.at[pl.ds(t * rpt, rpt)],
                        acc.at[pl.ds(t * rpt, rpt)])
        plsc.subcore_barrier()
        base = t * per_tile // 128
        off = c * nsrc

        @pl.loop(0, ngrp)
        def _(g):
            pltpu.sync_copy(src_hbm.at[pl.ds(base + g * 8, 8)], src_v)
            pltpu.sync_copy(dst_hbm.at[pl.ds(base + g * 8, 8)], dst_v)
            pltpu.sync_copy(w_hbm.at[pl.ds(base + g * 8, 8)], w_v)

            @pl.loop(0, 8)
            def _(j):
                for kk in range(8):
                    sl = pl.ds(kk * 16, 16)
                    src_v.at[j][sl] = src_v.at[j][sl] + off
                pltpu.async_copy(tab_hbm.at[src_v.at[j]], rows_v, sem).wait()
                for e16 in range(8):
                    wv = w_v.at[j][pl.ds(e16 * 16, 16)]
                    for q in range(16):
                        e = e16 * 16 + q
                        rows_v.at[e][...] = (
                            rows_v.at[e][...] * _lane_bcast(wv, q))
                pltpu.sync_copy(rows_v, acc.at[dst_v.at[j]], add=True)

        plsc.subcore_barrier()
        pltpu.sync_copy(acc.at[pl.ds(t * rpt, rpt)],
                        out_hbm.at[pl.ds(c * nr + t * rpt, rpt)])

    return k(table, src2d, dst2d, w2d, zeros)


_NH_SEG = 102400
_NS_SEG = 10112


def _sc_segsum(x2stk, seg2d):
    per_tile = _NH_SEG // _NS
    rpt = _NS_SEG // _NS
    zeros = jnp.zeros((_NS_SEG, 16), jnp.float32)

    @functools.partial(
        pl.kernel,
        out_type=jax.ShapeDtypeStruct((2 * _NS_SEG, 16), jnp.float32),
        mesh=_vmesh(),
        scratch_types=[
            pltpu.VMEM((5, 128), jnp.int32),
            pltpu.VMEM((640, 16), jnp.float32),
            pltpu.VMEM_SHARED((_NS_SEG, 16), jnp.float32),
        ],
        compiler_params=_SC_PARAMS,
    )
    def k(x_hbm, seg_hbm, zeros_hbm, out_hbm, seg_v, rows_v, acc):
        c = lax.axis_index("c")
        t = lax.axis_index("s")
        pltpu.sync_copy(zeros_hbm.at[pl.ds(t * rpt, rpt)],
                        acc.at[pl.ds(t * rpt, rpt)])
        plsc.subcore_barrier()
        base_r = c * _NH_SEG + t * per_tile
        base_s = t * (per_tile // 128)

        @pl.loop(0, 10)
        def _(g):
            pltpu.sync_copy(x_hbm.at[pl.ds(base_r + g * 640, 640)], rows_v)
            pltpu.sync_copy(seg_hbm.at[pl.ds(base_s + g * 5, 5)], seg_v)
            for j in range(5):
                pltpu.sync_copy(rows_v.at[pl.ds(j * 128, 128)],
                                acc.at[seg_v.at[j]], add=True)

        plsc.subcore_barrier()
        pltpu.sync_copy(acc.at[pl.ds(t * rpt, rpt)],
                        out_hbm.at[pl.ds(c * _NS_SEG + t * rpt, rpt)])

    return k(x2stk, seg2d, zeros)


def _sc_gather_scatter(table, src2d, dst2d, nr, d):
    return _sc_msg_pass(table, src2d, dst2d, None, nr, d, gsz=256)


def _pad_idx(a, e_pad, fill):
    return jnp.concatenate(
        [a, jnp.full((e_pad - a.shape[0],), fill, jnp.int32)]).reshape(-1, 128)


_EP_FA = 1605632
_NP_FA = 100096
_EP_RGCN = 327680
_NR_RGCN = 20096


def kernel(num_prop_burst, cat_prop_burst, tweet_range_list, edge_index_burst,
           re_index, des, tweet, num_prop, cat_prop, edge_index_rgcn,
           edge_type, params):
    p = params
    act = jax.nn.leaky_relu

    num = act(num_prop_burst @ p['w_num'] + p['b_num'])
    cat = act(cat_prop_burst @ p['w_cat'] + p['b_cat'])
    x = jnp.concatenate([num, cat], axis=1)
    x = act(x @ p['w_tog'] + p['b_tog'])

    src_b = _pad_idx(edge_index_burst[0], _EP_FA, 0)
    dst_b = _pad_idx(edge_index_burst[1], _EP_FA, 100000)
    deg2 = _sc_count(dst_b, _NP_FA)
    deg = deg2[:_NP_FA, 0] + deg2[_NP_FA:, 0]
    dis = jnp.where(deg > 0, lax.rsqrt(jnp.maximum(deg, 1.0)), 0.0)
    dis_e = _sc_edge_map(
        dis, [src_b, dst_b],
        lambda tab, v: (plsc.load_gather(tab, [v[0]])
                        * plsc.load_gather(tab, [v[1]])))

    def faconv_layer(xin, x0):
        l_pad = jnp.concatenate(
            [xin @ p['w_att_l'], jnp.zeros((_NP_FA - 100000,), jnp.float32)])
        r_pad = jnp.concatenate(
            [xin @ p['w_att_r'], jnp.zeros((_NP_FA - 100000,), jnp.float32)])
        ls = _sc_edge_map(l_pad, [src_b],
                          lambda tab, v: plsc.load_gather(tab, [v[0]]))

        def wfn(tab, v):
            z = v[1] + plsc.load_gather(tab, [v[0]])
            return (1.0 - 2.0 / (jnp.exp(2.0 * z) + 1.0)) * v[2]

        w2d = _sc_edge_map(r_pad, [dst_b, ls, dis_e], wfn)
        xstk = jnp.concatenate([
            jnp.pad(xin[:, :16], ((0, _NP_FA - 100000), (0, 0))),
            jnp.pad(xin[:, 16:], ((0, _NP_FA - 100000), (0, 0)))], axis=0)
        out = _sc_msg_pass(xstk, src_b, dst_b, w2d, _NP_FA, 16, gsz=512)
        o = jnp.concatenate(
            [out[:100000], out[_NP_FA:_NP_FA + 100000]], axis=1)
        return o + _EPS_FA * x0

    x1 = faconv_layer(x, x)
    x2 = faconv_layer(x1, x)
    x2 = (x2 ** 2 + 1e-08) ** 0.5

    hist = jnp.zeros((100000,), jnp.int32).at[tweet_range_list].add(1)
    seg = jnp.cumsum(hist) - 1
    seg = jnp.where((seg >= 0) & (seg < 10000), seg, 10000)
    seg2d = jnp.concatenate(
        [seg, jnp.full((_NH_SEG - 100000,), 10000, jnp.int32)]).reshape(-1, 128)
    x2stk = jnp.concatenate([
        jnp.pad(x2[:, :16], ((0, _NH_SEG - 100000), (0, 0))),
        jnp.pad(x2[:, 16:], ((0, _NH_SEG - 100000), (0, 0)))], axis=0)
    segsum = _sc_segsum(x2stk, seg2d)
    x3 = jnp.concatenate(
        [segsum[:10000], segsum[_NS_SEG:_NS_SEG + 10000]], axis=1)

    x3 = _sc_gather_rows(x3, re_index)
    x_burst = act(x3 @ p['w_map'] + p['b_map'])

    d = act(des @ p['w_des'] + p['b_des'])
    t = act(tweet @ p['w_tw'] + p['b_tw'])
    n = act(num_prop @ p['w_np'] + p['b_np'])
    c = act(cat_prop @ p['w_cp'] + p['b_cp'])
    xr = jnp.concatenate([d, t, n, c], axis=1)
    xr = act(xr @ p['w_in'] + p['b_in'])

    src_r = _pad_idx(edge_index_rgcn[0], _EP_RGCN, 0)
    gidx_r = _pad_idx(edge_index_rgcn[1] + edge_type * 10000, _EP_RGCN, 20000)
    cnt2 = _sc_count(gidx_r, _NR_RGCN)
    cnt = cnt2[:_NR_RGCN, 0] + cnt2[_NR_RGCN:, 0]
    c0 = jnp.maximum(cnt[:10000], 1.0)
    c1 = jnp.maximum(cnt[10000:20000], 1.0)

    def rgcn_layer(xin):
        xstk = jnp.concatenate([xin[:, :64], xin[:, 64:]], axis=0)
        s = _sc_gather_scatter(xstk, src_r, gidx_r, _NR_RGCN, 64)
        s_a, s_b = s[:_NR_RGCN], s[_NR_RGCN:]
        s0 = jnp.concatenate([s_a[:10000], s_b[:10000]], axis=1)
        s1 = jnp.concatenate([s_a[10000:20000], s_b[10000:20000]], axis=1)
        return (xin @ p['w_root'] + p['b_rgcn']
                + (s0 / c0[:, None]) @ p['w_rgcn'][0]
                + (s1 / c1[:, None]) @ p['w_rgcn'][1])

    xr = rgcn_layer(xr)
    xr = rgcn_layer(xr)
    x_rgcn = act(xr @ p['w_out1'] + p['b_out1'])

    xcat = jnp.concatenate([x_burst, x_rgcn], axis=1)
    xcat = act(xcat @ p['w_f0'] + p['b_f0'])
    return xcat @ p['w_f'] + p['b_f']
